# Initial kernel scaffold; baseline (speedup 1.0000x reference)
import jax
import jax.numpy as jnp
from jax.experimental import pallas as pl


def kernel(fire_features, weather_features, topo_features, edge_index, fw1, fb1, fg1, fbe1, fw2, fb2, ww1, wb1, wg1, wbe1, ww2, wb2, tw1, tb1, tg1, tbe1, tw2, tb2, fus_w, fus_b, g0_w, g0_b, g0_gamma, g0_beta, g1_w, g1_b, g1_gamma, g1_beta, g2_w, g2_b, g2_gamma, g2_beta, l0_wih, l0_bih, l0_whh, l0_bhh, l1_wih, l1_bih, l1_whh, l1_bhh, occ_w1, occ_b1, occ_w2, occ_b2, int_w1, int_b1, int_w2, int_b2):
    raise NotImplementedError("write your pallas kernel here")


# speedup vs baseline: 7.7960x; 7.7960x over previous
import functools

import jax
import jax.numpy as jnp
from jax import lax
from jax.experimental import pallas as pl
from jax.experimental.pallas import tpu as pltpu
from jax.experimental.pallas import tpu_sc as plsc

N = 50000
T = 4
E = 800000
HID = 64
EPAD = 802816
ACC_R = 50048
RPT = ACC_R // 16
DUMMY = N
BLK = 2000
GRID = N // BLK
EPS = 1e-5

_SC_CACHE = {}


def _sc_mesh():
    return plsc.VectorSubcoreMesh(core_axis_name="c", subcore_axis_name="s")


def _mmt(x, w):
    return lax.dot_general(x, w, (((1,), (1,)), ((), ())),
                           preferred_element_type=jnp.float32)


def _relu(x):
    return jnp.maximum(x, 0.0)


def _sigmoid(x):
    return 1.0 / (1.0 + jnp.exp(-x))


def _deg_kernel_body(dst_hbm, ones_hbm, zeros_hbm, out_hbm, idx_d, ones_v, acc):
    c = lax.axis_index("c")
    s = lax.axis_index("s")
    wid = s * 2 + c
    pltpu.sync_copy(zeros_hbm.at[pl.ds(s * RPT, RPT)], acc.at[pl.ds(s * RPT, RPT)])
    pltpu.sync_copy(ones_hbm, ones_v)
    plsc.subcore_barrier()

    @pl.loop(0, 28)
    def _blk(k):
        pltpu.sync_copy(dst_hbm.at[wid, k], idx_d)
        for j in range(7):
            pltpu.sync_copy(ones_v, acc.at[idx_d.at[j]], add=True)

    plsc.subcore_barrier()
    pltpu.sync_copy(acc.at[pl.ds(s * RPT, RPT)], out_hbm.at[c, pl.ds(s * RPT, RPT)])


def _spmm_kernel_body(table_hbm, idx8_hbm, dst_hbm, zeros_hbm, out_hbm,
                      idx_s, idx_d, rows, acc, gsem):
    c = lax.axis_index("c")
    s = lax.axis_index("s")
    for t in range(T):
        pltpu.sync_copy(zeros_hbm.at[pl.ds(s * RPT, RPT)],
                        acc.at[pl.ds(s * RPT, RPT)])
        plsc.subcore_barrier()

        @pl.loop(0, 56)
        def _blk(k):
            pltpu.sync_copy(idx8_hbm.at[t, c, s, k], idx_s)
            pltpu.sync_copy(dst_hbm.at[s, k], idx_d)
            for j in range(7):
                pltpu.async_copy(table_hbm.at[idx_s.at[j]], rows, gsem).wait()
                pltpu.sync_copy(rows, acc.at[idx_d.at[j]], add=True)

        plsc.subcore_barrier()
        pltpu.sync_copy(acc.at[pl.ds(s * RPT, RPT)],
                        out_hbm.at[t, c, pl.ds(s * RPT, RPT)])
        plsc.subcore_barrier()


def _sc_degree(dst32, ones_rows, zeros_acc):
    if "deg" not in _SC_CACHE:
        _SC_CACHE["deg"] = pl.kernel(
            _deg_kernel_body,
            out_type=jax.ShapeDtypeStruct((2, ACC_R, 32), jnp.float32),
            mesh=_sc_mesh(),
            scratch_types=[
                pltpu.VMEM((7, 128), jnp.int32),
                pltpu.VMEM((128, 32), jnp.float32),
                pltpu.VMEM_SHARED((ACC_R, 32), jnp.float32),
            ],
            compiler_params=pltpu.CompilerParams(use_tc_tiling_on_sc=False),
        )
    return _SC_CACHE["deg"](dst32, ones_rows, zeros_acc)


def _sc_spmm(table, idx8, dst16, zeros_acc):
    if "spmm" not in _SC_CACHE:
        _SC_CACHE["spmm"] = pl.kernel(
            _spmm_kernel_body,
            out_type=jax.ShapeDtypeStruct((T, 2, ACC_R, 32), jnp.float32),
            mesh=_sc_mesh(),
            scratch_types=[
                pltpu.VMEM((7, 128), jnp.int32),
                pltpu.VMEM((7, 128), jnp.int32),
                pltpu.VMEM((128, 32), jnp.float32),
                pltpu.VMEM_SHARED((ACC_R, 32), jnp.float32),
                pltpu.SemaphoreType.DMA,
            ],
            compiler_params=pltpu.CompilerParams(use_tc_tiling_on_sc=False),
        )
    return _SC_CACHE["spmm"](table, idx8, dst16, zeros_acc)


def _full_spec(a):
    nd = a.ndim
    return pl.BlockSpec(a.shape, lambda i, _nd=nd: (0,) * _nd)


def _encstats_body(fire, weath, topo, fw1, fb1, ww1, wb1, tw1, tb1, s1, s2):
    i = pl.program_id(0)

    @pl.when(i == 0)
    def _init():
        s1[...] = jnp.zeros_like(s1)
        s2[...] = jnp.zeros_like(s2)

    sa, sb = [], []
    for t in range(T):
        hf = _relu(_mmt(fire[:, t, :], fw1[...]) + fb1[...])
        hw = _relu(_mmt(weath[:, t, :], ww1[...]) + wb1[...])
        ht = _relu(_mmt(topo[:, t, :], tw1[...]) + tb1[...])
        cat = jnp.concatenate([hf, hw, ht], axis=1)
        sa.append(jnp.sum(cat, axis=0))
        sb.append(jnp.sum(cat * cat, axis=0))
    s1[...] += jnp.stack(sa)
    s2[...] += jnp.stack(sb)


def _enc_stats(fire, weath, topo, fw1, fb1, ww1, wb1, tw1, tb1):
    args = (fire, weath, topo, fw1, fb1, ww1, wb1, tw1, tb1)
    return pl.pallas_call(
        _encstats_body,
        grid=(GRID,),
        in_specs=[pl.BlockSpec((BLK, T, 10), lambda i: (i, 0, 0)),
                  pl.BlockSpec((BLK, T, 8), lambda i: (i, 0, 0)),
                  pl.BlockSpec((BLK, T, 9), lambda i: (i, 0, 0))]
                 + [_full_spec(a) for a in args[3:]],
        out_specs=[pl.BlockSpec((T, 96), lambda i: (0, 0))] * 2,
        out_shape=[jax.ShapeDtypeStruct((T, 96), jnp.float32)] * 2,
    )(*args)


def _encapply_body(fire, weath, topo,
                   fw1, fb1, fg1, fbe1, fw2, fb2,
                   ww1, wb1, wg1, wbe1, ww2, wb2,
                   tw1, tb1, tg1, tbe1, tw2, tb2,
                   fus_w, fus_b, s1, s2, out):
    s1v = s1[...]
    s2v = s2[...]
    gcat = jnp.concatenate([fg1[...], wg1[...], tg1[...]])
    bcat = jnp.concatenate([fbe1[...], wbe1[...], tbe1[...]])
    for t in range(T):
        hf = _relu(_mmt(fire[:, t, :], fw1[...]) + fb1[...])
        hw = _relu(_mmt(weath[:, t, :], ww1[...]) + wb1[...])
        ht = _relu(_mmt(topo[:, t, :], tw1[...]) + tb1[...])
        cat = jnp.concatenate([hf, hw, ht], axis=1)
        mean = s1v[t] / N
        var = s2v[t] / N - mean * mean
        scale = gcat * lax.rsqrt(var + EPS)
        shift = bcat - mean * scale
        xn = cat * scale + shift
        ef = _mmt(xn[:, 0:32], fw2[...]) + fb2[...]
        ew = _mmt(xn[:, 32:64], ww2[...]) + wb2[...]
        et = _mmt(xn[:, 64:96], tw2[...]) + tb2[...]
        cat2 = jnp.concatenate([ef, ew, et], axis=1)
        out[:, t, :] = _mmt(cat2, fus_w[...]) + fus_b[...]


def _enc_apply(fire, weath, topo, encw, fus_w, fus_b, s1, s2):
    args = (fire, weath, topo) + tuple(encw) + (fus_w, fus_b, s1, s2)
    return pl.pallas_call(
        _encapply_body,
        grid=(GRID,),
        in_specs=[pl.BlockSpec((BLK, T, 10), lambda i: (i, 0, 0)),
                  pl.BlockSpec((BLK, T, 8), lambda i: (i, 0, 0)),
                  pl.BlockSpec((BLK, T, 9), lambda i: (i, 0, 0))]
                 + [_full_spec(a) for a in args[3:]],
        out_specs=pl.BlockSpec((BLK, T, HID), lambda i: (i, 0, 0)),
        out_shape=jax.ShapeDtypeStruct((N, T, HID), jnp.float32),
    )(*args)


def _pre0_body(x, dinv, w, out):
    dv = dinv[...][:, 0]
    wv = w[...]
    for t in range(T):
        out[:, t, :] = _mmt(x[:, t, :], wv) * dv[:, None]


def _pre_l0(x, dinv, w):
    return pl.pallas_call(
        _pre0_body,
        grid=(GRID,),
        in_specs=[pl.BlockSpec((BLK, T, HID), lambda i: (i, 0, 0)),
                  pl.BlockSpec((BLK, 1), lambda i: (i, 0)),
                  _full_spec(w)],
        out_specs=pl.BlockSpec((BLK, T, HID), lambda i: (i, 0, 0)),
        out_shape=jax.ShapeDtypeStruct((N, T, HID), jnp.float32),
    )(x, dinv, w)


def _pre_body(x, s1, s2, gamma, beta, dinv, w, out):
    s1v = s1[...]
    s2v = s2[...]
    gv = gamma[...]
    bv = beta[...]
    dv = dinv[...][:, 0]
    wv = w[...]
    for t in range(T):
        mean = s1v[t] / N
        var = s2v[t] / N - mean * mean
        scale = gv * lax.rsqrt(var + EPS)
        shift = bv - mean * scale
        xt = _relu(x[:, t, :] * scale + shift)
        out[:, t, :] = _mmt(xt, wv) * dv[:, None]


def _pre_l(x, s1, s2, gamma, beta, dinv, w):
    args = (x, s1, s2, gamma, beta, dinv, w)
    return pl.pallas_call(
        _pre_body,
        grid=(GRID,),
        in_specs=[pl.BlockSpec((BLK, T, HID), lambda i: (i, 0, 0)),
                  _full_spec(s1), _full_spec(s2), _full_spec(gamma),
                  _full_spec(beta),
                  pl.BlockSpec((BLK, 1), lambda i: (i, 0)),
                  _full_spec(w)],
        out_specs=pl.BlockSpec((BLK, T, HID), lambda i: (i, 0, 0)),
        out_shape=jax.ShapeDtypeStruct((N, T, HID), jnp.float32),
    )(*args)


def _post_body(acc, gpre, dinv, b, out, s1, s2):
    i = pl.program_id(0)

    @pl.when(i == 0)
    def _init():
        s1[...] = jnp.zeros_like(s1)
        s2[...] = jnp.zeros_like(s2)

    dv = dinv[...][:, 0]
    bv = b[...]
    sa, sb = [], []
    for t in range(T):
        a = jnp.concatenate([acc[t, 0, :, :], acc[t, 1, :, :]], axis=1)
        r = dv[:, None] * (a + gpre[:, t, :]) + bv
        out[:, t, :] = r
        sa.append(jnp.sum(r, axis=0))
        sb.append(jnp.sum(r * r, axis=0))
    s1[...] += jnp.stack(sa)
    s2[...] += jnp.stack(sb)


def _post_l(acc, gpre, dinv, b):
    return pl.pallas_call(
        _post_body,
        grid=(GRID,),
        in_specs=[pl.BlockSpec((T, 2, BLK, 32), lambda i: (0, 0, i, 0)),
                  pl.BlockSpec((BLK, T, HID), lambda i: (i, 0, 0)),
                  pl.BlockSpec((BLK, 1), lambda i: (i, 0)),
                  _full_spec(b)],
        out_specs=[pl.BlockSpec((BLK, T, HID), lambda i: (i, 0, 0)),
                   pl.BlockSpec((T, HID), lambda i: (0, 0)),
                   pl.BlockSpec((T, HID), lambda i: (0, 0))],
        out_shape=[jax.ShapeDtypeStruct((N, T, HID), jnp.float32),
                   jax.ShapeDtypeStruct((T, HID), jnp.float32),
                   jax.ShapeDtypeStruct((T, HID), jnp.float32)],
    )(acc, gpre, dinv, b)


def _lstm_body(x, s1, s2, gamma, beta,
               wih0, bih0, whh0, bhh0, wih1, bih1, whh1, bhh1,
               ow1, ob1, ow2, ob2, iw1, ib1, iw2, ib2,
               occ_out, int_out):
    s1v = s1[...]
    s2v = s2[...]
    gv = gamma[...]
    bv = beta[...]
    xs = []
    for t in range(T):
        mean = s1v[t] / N
        var = s2v[t] / N - mean * mean
        scale = gv * lax.rsqrt(var + EPS)
        shift = bv - mean * scale
        xs.append(_relu(x[:, t, :] * scale + shift))

    def lstm(seq, wih, bih, whh, bhh):
        h = jnp.zeros((BLK, HID), jnp.float32)
        cc = jnp.zeros((BLK, HID), jnp.float32)
        hs = []
        for t in range(T):
            gates = (_mmt(seq[t], wih[...]) + bih[...]
                     + _mmt(h, whh[...]) + bhh[...])
            ig = _sigmoid(gates[:, 0:HID])
            fg = _sigmoid(gates[:, HID:2 * HID])
            gg = jnp.tanh(gates[:, 2 * HID:3 * HID])
            og = _sigmoid(gates[:, 3 * HID:4 * HID])
            cc = fg * cc + ig * gg
            h = og * jnp.tanh(cc)
            hs.append(h)
        return hs

    h1 = lstm(xs, wih0, bih0, whh0, bhh0)
    h2 = lstm(h1, wih1, bih1, whh1, bhh1)
    final = h2[-1]
    ho = _relu(_mmt(final, ow1[...]) + ob1[...])
    occ = _sigmoid(jnp.sum(ho * ow2[...][0][None, :], axis=1, keepdims=True)
                   + ob2[...][0])
    hi = _relu(_mmt(final, iw1[...]) + ib1[...])
    inten = (jnp.sum(hi * iw2[...][0][None, :], axis=1, keepdims=True)
             + ib2[...][0])
    occ_out[...] = occ
    int_out[...] = inten


def _lstm_heads(x, s1, s2, gamma, beta, lstm_w, head_w):
    args = (x, s1, s2, gamma, beta) + tuple(lstm_w) + tuple(head_w)
    return pl.pallas_call(
        _lstm_body,
        grid=(GRID,),
        in_specs=[pl.BlockSpec((BLK, T, HID), lambda i: (i, 0, 0))]
                 + [_full_spec(a) for a in args[1:]],
        out_specs=[pl.BlockSpec((BLK, 1), lambda i: (i, 0))] * 2,
        out_shape=[jax.ShapeDtypeStruct((N, 1), jnp.float32)] * 2,
    )(*args)


def _dinv_body(degs, out):
    d = degs[0, :, 0] + degs[1, :, 0] + 1.0
    out[...] = lax.rsqrt(d)[:, None]


def _dinv_tc(degs):
    return pl.pallas_call(
        _dinv_body,
        grid=(GRID,),
        in_specs=[pl.BlockSpec((2, BLK, 32), lambda i: (0, i, 0))],
        out_specs=pl.BlockSpec((BLK, 1), lambda i: (i, 0)),
        out_shape=jax.ShapeDtypeStruct((N, 1), jnp.float32),
    )(degs)


def kernel(fire_features, weather_features, topo_features, edge_index,
           fw1, fb1, fg1, fbe1, fw2, fb2,
           ww1, wb1, wg1, wbe1, ww2, wb2,
           tw1, tb1, tg1, tbe1, tw2, tb2,
           fus_w, fus_b,
           g0_w, g0_b, g0_gamma, g0_beta,
           g1_w, g1_b, g1_gamma, g1_beta,
           g2_w, g2_b, g2_gamma, g2_beta,
           l0_wih, l0_bih, l0_whh, l0_bhh,
           l1_wih, l1_bih, l1_whh, l1_bhh,
           occ_w1, occ_b1, occ_w2, occ_b2,
           int_w1, int_b1, int_w2, int_b2):
    src = edge_index[0]
    dst = edge_index[1]
    pad = EPAD - E
    src_p = jnp.concatenate([src, jnp.zeros((pad,), jnp.int32)])
    dst_p = jnp.concatenate([dst, jnp.full((pad,), DUMMY, jnp.int32)])
    dst32 = dst_p.reshape(32, 28, 7, 128)
    dst16 = dst_p.reshape(16, 56, 7, 128)
    base8 = src_p * 8
    idx8 = (base8[None, None, :]
            + (jnp.arange(T, dtype=jnp.int32) * 2)[:, None, None]
            + jnp.arange(2, dtype=jnp.int32)[None, :, None])
    idx8 = idx8.reshape(T, 2, 16, 56, 7, 128)
    zeros_acc = jnp.zeros((ACC_R, 32), jnp.float32)
    ones_rows = jnp.ones((128, 32), jnp.float32)

    degs = _sc_degree(dst32, ones_rows, zeros_acc)
    dinv = _dinv_tc(degs)

    s1e, s2e = _enc_stats(fire_features, weather_features, topo_features,
                          fw1, fb1, ww1, wb1, tw1, tb1)
    encw = (fw1, fb1, fg1, fbe1, fw2, fb2,
            ww1, wb1, wg1, wbe1, ww2, wb2,
            tw1, tb1, tg1, tbe1, tw2, tb2)
    h0 = _enc_apply(fire_features, weather_features, topo_features,
                    encw, fus_w, fus_b, s1e, s2e)

    gws = (g0_w, g1_w, g2_w)
    gbs = (g0_b, g1_b, g2_b)
    ggammas = (g0_gamma, g1_gamma, g2_gamma)
    gbetas = (g0_beta, g1_beta, g2_beta)

    raw, s1, s2 = None, None, None
    for l in range(3):
        if l == 0:
            gpre = _pre_l0(h0, dinv, gws[l])
        else:
            gpre = _pre_l(raw, s1, s2, ggammas[l - 1], gbetas[l - 1],
                          dinv, gws[l])
        acc = _sc_spmm(gpre.reshape(N * 8, 32), idx8, dst16, zeros_acc)
        raw, s1, s2 = _post_l(acc, gpre, dinv, gbs[l])

    lstm_w = (l0_wih, l0_bih, l0_whh, l0_bhh, l1_wih, l1_bih, l1_whh, l1_bhh)
    head_w = (occ_w1, occ_b1, occ_w2, occ_b2, int_w1, int_b1, int_w2, int_b2)
    occ, inten = _lstm_heads(raw, s1, s2, g2_gamma, g2_beta, lstm_w, head_w)
    return occ, inten


# speedup vs baseline: 10.6958x; 1.3720x over previous
import functools

import jax
import jax.numpy as jnp
from jax import lax
from jax.experimental import pallas as pl
from jax.experimental.pallas import tpu as pltpu
from jax.experimental.pallas import tpu_sc as plsc

N = 50000
T = 4
E = 800000
HID = 64
EPAD = 802816
ACC_R = 50048
RPT = ACC_R // 16
DUMMY = N
BLK = 2000
GRID = N // BLK
EPS = 1e-5

_SC_CACHE = {}


def _sc_mesh():
    return plsc.VectorSubcoreMesh(core_axis_name="c", subcore_axis_name="s")


def _mmt(x, w):
    return lax.dot_general(x, w, (((1,), (1,)), ((), ())),
                           preferred_element_type=jnp.float32)


def _relu(x):
    return jnp.maximum(x, 0.0)


def _sigmoid(x):
    return 1.0 / (1.0 + jnp.exp(-x))


def _deg_kernel_body(dst_hbm, ones_hbm, zeros_hbm, out_hbm, idx_d, ones_v, acc):
    c = lax.axis_index("c")
    s = lax.axis_index("s")
    wid = s * 2 + c
    pltpu.sync_copy(zeros_hbm.at[pl.ds(s * RPT, RPT)], acc.at[pl.ds(s * RPT, RPT)])
    pltpu.sync_copy(ones_hbm, ones_v)
    plsc.subcore_barrier()

    @pl.loop(0, 28)
    def _blk(k):
        pltpu.sync_copy(dst_hbm.at[wid, k], idx_d)
        for j in range(7):
            pltpu.sync_copy(ones_v, acc.at[idx_d.at[j]], add=True)

    plsc.subcore_barrier()
    pltpu.sync_copy(acc.at[pl.ds(s * RPT, RPT)], out_hbm.at[c, pl.ds(s * RPT, RPT)])


def _spmm_kernel_body(table_hbm, cidx_hbm, zeros_hbm, out_hbm,
                      cidx_v, rows, acc, gsem, ssem):
    c = lax.axis_index("c")
    s = lax.axis_index("s")
    for t in range(T):
        pltpu.sync_copy(zeros_hbm.at[pl.ds(s * RPT, RPT)],
                        acc.at[pl.ds(s * RPT, RPT)])
        plsc.subcore_barrier()

        @pl.loop(0, 98)
        def _blk(k):
            pltpu.sync_copy(cidx_hbm.at[t, c, s, k], cidx_v)
            hs = [pltpu.async_copy(table_hbm.at[cidx_v.at[0, j]],
                                   rows.at[j], gsem.at[j])
                  for j in range(4)]
            sh = []
            for j in range(4):
                hs[j].wait()
                sh.append(pltpu.async_copy(rows.at[j], acc.at[cidx_v.at[1, j]],
                                           ssem.at[j], add=True))
            for h in sh:
                h.wait()

        plsc.subcore_barrier()
        pltpu.sync_copy(acc.at[pl.ds(s * RPT, RPT)],
                        out_hbm.at[t, c, pl.ds(s * RPT, RPT)])
        plsc.subcore_barrier()


def _sc_degree(dst32, ones_rows, zeros_acc):
    if "deg" not in _SC_CACHE:
        _SC_CACHE["deg"] = pl.kernel(
            _deg_kernel_body,
            out_type=jax.ShapeDtypeStruct((2, ACC_R, 32), jnp.float32),
            mesh=_sc_mesh(),
            scratch_types=[
                pltpu.VMEM((7, 128), jnp.int32),
                pltpu.VMEM((128, 32), jnp.float32),
                pltpu.VMEM_SHARED((ACC_R, 32), jnp.float32),
            ],
            compiler_params=pltpu.CompilerParams(use_tc_tiling_on_sc=False),
        )
    return _SC_CACHE["deg"](dst32, ones_rows, zeros_acc)


def _sc_spmm(table, cidx, zeros_acc):
    if "spmm" not in _SC_CACHE:
        _SC_CACHE["spmm"] = pl.kernel(
            _spmm_kernel_body,
            out_type=jax.ShapeDtypeStruct((T, 2, ACC_R, 32), jnp.float32),
            mesh=_sc_mesh(),
            scratch_types=[
                pltpu.VMEM((2, 4, 128), jnp.int32),
                pltpu.VMEM((4, 128, 32), jnp.float32),
                pltpu.VMEM_SHARED((ACC_R, 32), jnp.float32),
                pltpu.SemaphoreType.DMA((4,)),
                pltpu.SemaphoreType.DMA((4,)),
            ],
            compiler_params=pltpu.CompilerParams(use_tc_tiling_on_sc=False),
        )
    return _SC_CACHE["spmm"](table, cidx, zeros_acc)


def _full_spec(a):
    nd = a.ndim
    return pl.BlockSpec(a.shape, lambda i, _nd=nd: (0,) * _nd)


def _encstats_body(fire, weath, topo, fw1, fb1, ww1, wb1, tw1, tb1, s1, s2):
    i = pl.program_id(0)

    @pl.when(i == 0)
    def _init():
        s1[...] = jnp.zeros_like(s1)
        s2[...] = jnp.zeros_like(s2)

    sa, sb = [], []
    for t in range(T):
        hf = _relu(_mmt(fire[:, t, :], fw1[...]) + fb1[...])
        hw = _relu(_mmt(weath[:, t, :], ww1[...]) + wb1[...])
        ht = _relu(_mmt(topo[:, t, :], tw1[...]) + tb1[...])
        cat = jnp.concatenate([hf, hw, ht], axis=1)
        sa.append(jnp.sum(cat, axis=0))
        sb.append(jnp.sum(cat * cat, axis=0))
    s1[...] += jnp.stack(sa)
    s2[...] += jnp.stack(sb)


def _enc_stats(fire, weath, topo, fw1, fb1, ww1, wb1, tw1, tb1):
    args = (fire, weath, topo, fw1, fb1, ww1, wb1, tw1, tb1)
    return pl.pallas_call(
        _encstats_body,
        grid=(GRID,),
        in_specs=[pl.BlockSpec((BLK, T, 10), lambda i: (i, 0, 0)),
                  pl.BlockSpec((BLK, T, 8), lambda i: (i, 0, 0)),
                  pl.BlockSpec((BLK, T, 9), lambda i: (i, 0, 0))]
                 + [_full_spec(a) for a in args[3:]],
        out_specs=[pl.BlockSpec((T, 96), lambda i: (0, 0))] * 2,
        out_shape=[jax.ShapeDtypeStruct((T, 96), jnp.float32)] * 2,
    )(*args)


def _encapply_body(fire, weath, topo,
                   fw1, fb1, fg1, fbe1, fw2, fb2,
                   ww1, wb1, wg1, wbe1, ww2, wb2,
                   tw1, tb1, tg1, tbe1, tw2, tb2,
                   fus_w, fus_b, s1, s2, out):
    s1v = s1[...]
    s2v = s2[...]
    gcat = jnp.concatenate([fg1[...], wg1[...], tg1[...]])
    bcat = jnp.concatenate([fbe1[...], wbe1[...], tbe1[...]])
    for t in range(T):
        hf = _relu(_mmt(fire[:, t, :], fw1[...]) + fb1[...])
        hw = _relu(_mmt(weath[:, t, :], ww1[...]) + wb1[...])
        ht = _relu(_mmt(topo[:, t, :], tw1[...]) + tb1[...])
        cat = jnp.concatenate([hf, hw, ht], axis=1)
        mean = s1v[t] / N
        var = s2v[t] / N - mean * mean
        scale = gcat * lax.rsqrt(var + EPS)
        shift = bcat - mean * scale
        xn = cat * scale + shift
        ef = _mmt(xn[:, 0:32], fw2[...]) + fb2[...]
        ew = _mmt(xn[:, 32:64], ww2[...]) + wb2[...]
        et = _mmt(xn[:, 64:96], tw2[...]) + tb2[...]
        cat2 = jnp.concatenate([ef, ew, et], axis=1)
        out[:, t, :] = _mmt(cat2, fus_w[...]) + fus_b[...]


def _enc_apply(fire, weath, topo, encw, fus_w, fus_b, s1, s2):
    args = (fire, weath, topo) + tuple(encw) + (fus_w, fus_b, s1, s2)
    return pl.pallas_call(
        _encapply_body,
        grid=(GRID,),
        in_specs=[pl.BlockSpec((BLK, T, 10), lambda i: (i, 0, 0)),
                  pl.BlockSpec((BLK, T, 8), lambda i: (i, 0, 0)),
                  pl.BlockSpec((BLK, T, 9), lambda i: (i, 0, 0))]
                 + [_full_spec(a) for a in args[3:]],
        out_specs=pl.BlockSpec((BLK, T, HID), lambda i: (i, 0, 0)),
        out_shape=jax.ShapeDtypeStruct((N, T, HID), jnp.float32),
    )(*args)


def _pre0_body(x, dinv, w, out):
    dv = dinv[...][:, 0]
    wv = w[...]
    for t in range(T):
        out[:, t, :] = _mmt(x[:, t, :], wv) * dv[:, None]


def _pre_l0(x, dinv, w):
    return pl.pallas_call(
        _pre0_body,
        grid=(GRID,),
        in_specs=[pl.BlockSpec((BLK, T, HID), lambda i: (i, 0, 0)),
                  pl.BlockSpec((BLK, 1), lambda i: (i, 0)),
                  _full_spec(w)],
        out_specs=pl.BlockSpec((BLK, T, HID), lambda i: (i, 0, 0)),
        out_shape=jax.ShapeDtypeStruct((N, T, HID), jnp.float32),
    )(x, dinv, w)


def _pre_body(x, s1, s2, gamma, beta, dinv, w, out):
    s1v = s1[...]
    s2v = s2[...]
    gv = gamma[...]
    bv = beta[...]
    dv = dinv[...][:, 0]
    wv = w[...]
    for t in range(T):
        mean = s1v[t] / N
        var = s2v[t] / N - mean * mean
        scale = gv * lax.rsqrt(var + EPS)
        shift = bv - mean * scale
        xt = _relu(x[:, t, :] * scale + shift)
        out[:, t, :] = _mmt(xt, wv) * dv[:, None]


def _pre_l(x, s1, s2, gamma, beta, dinv, w):
    args = (x, s1, s2, gamma, beta, dinv, w)
    return pl.pallas_call(
        _pre_body,
        grid=(GRID,),
        in_specs=[pl.BlockSpec((BLK, T, HID), lambda i: (i, 0, 0)),
                  _full_spec(s1), _full_spec(s2), _full_spec(gamma),
                  _full_spec(beta),
                  pl.BlockSpec((BLK, 1), lambda i: (i, 0)),
                  _full_spec(w)],
        out_specs=pl.BlockSpec((BLK, T, HID), lambda i: (i, 0, 0)),
        out_shape=jax.ShapeDtypeStruct((N, T, HID), jnp.float32),
    )(*args)


def _post_body(acc, gpre, dinv, b, out, s1, s2):
    i = pl.program_id(0)

    @pl.when(i == 0)
    def _init():
        s1[...] = jnp.zeros_like(s1)
        s2[...] = jnp.zeros_like(s2)

    dv = dinv[...][:, 0]
    bv = b[...]
    sa, sb = [], []
    for t in range(T):
        a = jnp.concatenate([acc[t, 0, :, :], acc[t, 1, :, :]], axis=1)
        r = dv[:, None] * (a + gpre[:, t, :]) + bv
        out[:, t, :] = r
        sa.append(jnp.sum(r, axis=0))
        sb.append(jnp.sum(r * r, axis=0))
    s1[...] += jnp.stack(sa)
    s2[...] += jnp.stack(sb)


def _post_l(acc, gpre, dinv, b):
    return pl.pallas_call(
        _post_body,
        grid=(GRID,),
        in_specs=[pl.BlockSpec((T, 2, BLK, 32), lambda i: (0, 0, i, 0)),
                  pl.BlockSpec((BLK, T, HID), lambda i: (i, 0, 0)),
                  pl.BlockSpec((BLK, 1), lambda i: (i, 0)),
                  _full_spec(b)],
        out_specs=[pl.BlockSpec((BLK, T, HID), lambda i: (i, 0, 0)),
                   pl.BlockSpec((T, HID), lambda i: (0, 0)),
                   pl.BlockSpec((T, HID), lambda i: (0, 0))],
        out_shape=[jax.ShapeDtypeStruct((N, T, HID), jnp.float32),
                   jax.ShapeDtypeStruct((T, HID), jnp.float32),
                   jax.ShapeDtypeStruct((T, HID), jnp.float32)],
    )(acc, gpre, dinv, b)


def _lstm_body(x, s1, s2, gamma, beta,
               wih0, bih0, whh0, bhh0, wih1, bih1, whh1, bhh1,
               ow1, ob1, ow2, ob2, iw1, ib1, iw2, ib2,
               occ_out, int_out):
    s1v = s1[...]
    s2v = s2[...]
    gv = gamma[...]
    bv = beta[...]
    xs = []
    for t in range(T):
        mean = s1v[t] / N
        var = s2v[t] / N - mean * mean
        scale = gv * lax.rsqrt(var + EPS)
        shift = bv - mean * scale
        xs.append(_relu(x[:, t, :] * scale + shift))

    def lstm(seq, wih, bih, whh, bhh):
        h = jnp.zeros((BLK, HID), jnp.float32)
        cc = jnp.zeros((BLK, HID), jnp.float32)
        hs = []
        for t in range(T):
            gates = (_mmt(seq[t], wih[...]) + bih[...]
                     + _mmt(h, whh[...]) + bhh[...])
            ig = _sigmoid(gates[:, 0:HID])
            fg = _sigmoid(gates[:, HID:2 * HID])
            gg = jnp.tanh(gates[:, 2 * HID:3 * HID])
            og = _sigmoid(gates[:, 3 * HID:4 * HID])
            cc = fg * cc + ig * gg
            h = og * jnp.tanh(cc)
            hs.append(h)
        return hs

    h1 = lstm(xs, wih0, bih0, whh0, bhh0)
    h2 = lstm(h1, wih1, bih1, whh1, bhh1)
    final = h2[-1]
    ho = _relu(_mmt(final, ow1[...]) + ob1[...])
    occ = _sigmoid(jnp.sum(ho * ow2[...][0][None, :], axis=1, keepdims=True)
                   + ob2[...][0])
    hi = _relu(_mmt(final, iw1[...]) + ib1[...])
    inten = (jnp.sum(hi * iw2[...][0][None, :], axis=1, keepdims=True)
             + ib2[...][0])
    occ_out[...] = occ
    int_out[...] = inten


def _lstm_heads(x, s1, s2, gamma, beta, lstm_w, head_w):
    args = (x, s1, s2, gamma, beta) + tuple(lstm_w) + tuple(head_w)
    return pl.pallas_call(
        _lstm_body,
        grid=(GRID,),
        in_specs=[pl.BlockSpec((BLK, T, HID), lambda i: (i, 0, 0))]
                 + [_full_spec(a) for a in args[1:]],
        out_specs=[pl.BlockSpec((BLK, 1), lambda i: (i, 0))] * 2,
        out_shape=[jax.ShapeDtypeStruct((N, 1), jnp.float32)] * 2,
    )(*args)


def _dinv_body(degs, out):
    d = degs[0, :, 0] + degs[1, :, 0] + 1.0
    out[...] = lax.rsqrt(d)[:, None]


def _dinv_tc(degs):
    return pl.pallas_call(
        _dinv_body,
        grid=(GRID,),
        in_specs=[pl.BlockSpec((2, BLK, 32), lambda i: (0, i, 0))],
        out_specs=pl.BlockSpec((BLK, 1), lambda i: (i, 0)),
        out_shape=jax.ShapeDtypeStruct((N, 1), jnp.float32),
    )(degs)


def kernel(fire_features, weather_features, topo_features, edge_index,
           fw1, fb1, fg1, fbe1, fw2, fb2,
           ww1, wb1, wg1, wbe1, ww2, wb2,
           tw1, tb1, tg1, tbe1, tw2, tb2,
           fus_w, fus_b,
           g0_w, g0_b, g0_gamma, g0_beta,
           g1_w, g1_b, g1_gamma, g1_beta,
           g2_w, g2_b, g2_gamma, g2_beta,
           l0_wih, l0_bih, l0_whh, l0_bhh,
           l1_wih, l1_bih, l1_whh, l1_bhh,
           occ_w1, occ_b1, occ_w2, occ_b2,
           int_w1, int_b1, int_w2, int_b2):
    src = edge_index[0]
    dst = edge_index[1]
    pad = EPAD - E
    src_p = jnp.concatenate([src, jnp.zeros((pad,), jnp.int32)])
    dst_p = jnp.concatenate([dst, jnp.full((pad,), DUMMY, jnp.int32)])
    dst32 = dst_p.reshape(32, 28, 7, 128)
    base8 = src_p * 8
    src8 = (base8[None, None, :]
            + (jnp.arange(T, dtype=jnp.int32) * 2)[:, None, None]
            + jnp.arange(2, dtype=jnp.int32)[None, :, None])
    src8 = src8.reshape(T, 2, 16, 98, 4, 128)
    dstb = jnp.broadcast_to(dst_p.reshape(16, 98, 4, 128)[None, None],
                            (T, 2, 16, 98, 4, 128))
    cidx = jnp.stack([src8, dstb], axis=4)
    zeros_acc = jnp.zeros((ACC_R, 32), jnp.float32)
    ones_rows = jnp.ones((128, 32), jnp.float32)

    degs = _sc_degree(dst32, ones_rows, zeros_acc)
    dinv = _dinv_tc(degs)

    s1e, s2e = _enc_stats(fire_features, weather_features, topo_features,
                          fw1, fb1, ww1, wb1, tw1, tb1)
    encw = (fw1, fb1, fg1, fbe1, fw2, fb2,
            ww1, wb1, wg1, wbe1, ww2, wb2,
            tw1, tb1, tg1, tbe1, tw2, tb2)
    h0 = _enc_apply(fire_features, weather_features, topo_features,
                    encw, fus_w, fus_b, s1e, s2e)

    gws = (g0_w, g1_w, g2_w)
    gbs = (g0_b, g1_b, g2_b)
    ggammas = (g0_gamma, g1_gamma, g2_gamma)
    gbetas = (g0_beta, g1_beta, g2_beta)

    raw, s1, s2 = None, None, None
    for l in range(3):
        if l == 0:
            gpre = _pre_l0(h0, dinv, gws[l])
        else:
            gpre = _pre_l(raw, s1, s2, ggammas[l - 1], gbetas[l - 1],
                          dinv, gws[l])
        acc = _sc_spmm(gpre.reshape(N * 8, 32), cidx, zeros_acc)
        raw, s1, s2 = _post_l(acc, gpre, dinv, gbs[l])

    lstm_w = (l0_wih, l0_bih, l0_whh, l0_bhh, l1_wih, l1_bih, l1_whh, l1_bhh)
    head_w = (occ_w1, occ_b1, occ_w2, occ_b2, int_w1, int_b1, int_w2, int_b2)
    occ, inten = _lstm_heads(raw, s1, s2, g2_gamma, g2_beta, lstm_w, head_w)
    return occ, inten


# speedup vs baseline: 13.2715x; 1.2408x over previous
import functools

import jax
import jax.numpy as jnp
from jax import lax
from jax.experimental import pallas as pl
from jax.experimental.pallas import tpu as pltpu
from jax.experimental.pallas import tpu_sc as plsc

N = 50000
T = 4
E = 800000
HID = 64
EPAD = 802816
ACC_R = 50048
RPT = ACC_R // 16
DUMMY = N
BLK = 2000
GRID = N // BLK
EPS = 1e-5

_SC_CACHE = {}


def _sc_mesh():
    return plsc.VectorSubcoreMesh(core_axis_name="c", subcore_axis_name="s")


def _mmt(x, w):
    return lax.dot_general(x, w, (((1,), (1,)), ((), ())),
                           preferred_element_type=jnp.float32)


def _relu(x):
    return jnp.maximum(x, 0.0)


def _sigmoid(x):
    return 1.0 / (1.0 + jnp.exp(-x))


def _deg_kernel_body(dst_hbm, ones_hbm, zeros_hbm, out_hbm, idx_d, ones_v, acc):
    c = lax.axis_index("c")
    s = lax.axis_index("s")
    wid = s * 2 + c
    pltpu.sync_copy(zeros_hbm.at[pl.ds(s * RPT, RPT)], acc.at[pl.ds(s * RPT, RPT)])
    pltpu.sync_copy(ones_hbm, ones_v)
    plsc.subcore_barrier()

    @pl.loop(0, 28)
    def _blk(k):
        pltpu.sync_copy(dst_hbm.at[wid, k], idx_d)
        for j in range(7):
            pltpu.sync_copy(ones_v, acc.at[idx_d.at[j]], add=True)

    plsc.subcore_barrier()
    pltpu.sync_copy(acc.at[pl.ds(s * RPT, RPT)], out_hbm.at[c, pl.ds(s * RPT, RPT)])


def _spmm_kernel_body(table_hbm, cidx_hbm, zeros_hbm, out_hbm,
                      cidx_v, rows, acc, gsem, ssem):
    c = lax.axis_index("c")
    s = lax.axis_index("s")
    pltpu.sync_copy(zeros_hbm.at[pl.ds(s * RPT, RPT)],
                    acc.at[pl.ds(s * RPT, RPT)])
    plsc.subcore_barrier()

    @pl.loop(0, 98)
    def _blk(k):
        pltpu.sync_copy(cidx_hbm.at[c, s, k], cidx_v)
        hs = [pltpu.async_copy(table_hbm.at[cidx_v.at[0, j]],
                               rows.at[j], gsem.at[j])
              for j in range(4)]
        sh = []
        for j in range(4):
            hs[j].wait()
            sh.append(pltpu.async_copy(rows.at[j], acc.at[cidx_v.at[1, j]],
                                       ssem.at[j], add=True))
        for h in sh:
            h.wait()

    plsc.subcore_barrier()
    pltpu.sync_copy(acc.at[pl.ds(s * RPT, RPT)],
                    out_hbm.at[c, pl.ds(s * RPT, RPT)])


def _sc_degree(dst32, ones_rows, zeros_acc):
    if "deg" not in _SC_CACHE:
        _SC_CACHE["deg"] = pl.kernel(
            _deg_kernel_body,
            out_type=jax.ShapeDtypeStruct((2, ACC_R, 32), jnp.float32),
            mesh=_sc_mesh(),
            scratch_types=[
                pltpu.VMEM((7, 128), jnp.int32),
                pltpu.VMEM((128, 32), jnp.float32),
                pltpu.VMEM_SHARED((ACC_R, 32), jnp.float32),
            ],
            compiler_params=pltpu.CompilerParams(use_tc_tiling_on_sc=False),
        )
    return _SC_CACHE["deg"](dst32, ones_rows, zeros_acc)


def _sc_spmm(table, cidx, zeros_acc):
    if "spmm" not in _SC_CACHE:
        _SC_CACHE["spmm"] = pl.kernel(
            _spmm_kernel_body,
            out_type=jax.ShapeDtypeStruct((2, ACC_R, 32), jnp.float32),
            mesh=_sc_mesh(),
            scratch_types=[
                pltpu.VMEM((2, 4, 128), jnp.int32),
                pltpu.VMEM((4, 128, 32), jnp.float32),
                pltpu.VMEM_SHARED((ACC_R, 32), jnp.float32),
                pltpu.SemaphoreType.DMA((4,)),
                pltpu.SemaphoreType.DMA((4,)),
            ],
            compiler_params=pltpu.CompilerParams(use_tc_tiling_on_sc=False),
        )
    return _SC_CACHE["spmm"](table, cidx, zeros_acc)


def _full_spec(a):
    nd = a.ndim
    return pl.BlockSpec(a.shape, lambda i, _nd=nd: (0,) * _nd)


def _encstats_body(fire, weath, topo, fw1, fb1, ww1, wb1, tw1, tb1, s1, s2):
    i = pl.program_id(0)

    @pl.when(i == 0)
    def _init():
        s1[...] = jnp.zeros_like(s1)
        s2[...] = jnp.zeros_like(s2)

    sa, sb = [], []
    for t in range(T):
        hf = _relu(_mmt(fire[:, t, :], fw1[...]) + fb1[...])
        hw = _relu(_mmt(weath[:, t, :], ww1[...]) + wb1[...])
        ht = _relu(_mmt(topo[:, t, :], tw1[...]) + tb1[...])
        cat = jnp.concatenate([hf, hw, ht], axis=1)
        sa.append(jnp.sum(cat, axis=0))
        sb.append(jnp.sum(cat * cat, axis=0))
    s1[...] += jnp.stack(sa)
    s2[...] += jnp.stack(sb)


def _enc_stats(fire, weath, topo, fw1, fb1, ww1, wb1, tw1, tb1):
    args = (fire, weath, topo, fw1, fb1, ww1, wb1, tw1, tb1)
    return pl.pallas_call(
        _encstats_body,
        grid=(GRID,),
        in_specs=[pl.BlockSpec((BLK, T, 10), lambda i: (i, 0, 0)),
                  pl.BlockSpec((BLK, T, 8), lambda i: (i, 0, 0)),
                  pl.BlockSpec((BLK, T, 9), lambda i: (i, 0, 0))]
                 + [_full_spec(a) for a in args[3:]],
        out_specs=[pl.BlockSpec((T, 96), lambda i: (0, 0))] * 2,
        out_shape=[jax.ShapeDtypeStruct((T, 96), jnp.float32)] * 2,
    )(*args)


def _encapply_body(fire, weath, topo,
                   fw1, fb1, fg1, fbe1, fw2, fb2,
                   ww1, wb1, wg1, wbe1, ww2, wb2,
                   tw1, tb1, tg1, tbe1, tw2, tb2,
                   fus_w, fus_b, s1, s2, *outs):
    s1v = s1[...]
    s2v = s2[...]
    gcat = jnp.concatenate([fg1[...], wg1[...], tg1[...]])
    bcat = jnp.concatenate([fbe1[...], wbe1[...], tbe1[...]])
    for t in range(T):
        hf = _relu(_mmt(fire[:, t, :], fw1[...]) + fb1[...])
        hw = _relu(_mmt(weath[:, t, :], ww1[...]) + wb1[...])
        ht = _relu(_mmt(topo[:, t, :], tw1[...]) + tb1[...])
        cat = jnp.concatenate([hf, hw, ht], axis=1)
        mean = s1v[t] / N
        var = s2v[t] / N - mean * mean
        scale = gcat * lax.rsqrt(var + EPS)
        shift = bcat - mean * scale
        xn = cat * scale + shift
        ef = _mmt(xn[:, 0:32], fw2[...]) + fb2[...]
        ew = _mmt(xn[:, 32:64], ww2[...]) + wb2[...]
        et = _mmt(xn[:, 64:96], tw2[...]) + tb2[...]
        cat2 = jnp.concatenate([ef, ew, et], axis=1)
        outs[t][...] = _mmt(cat2, fus_w[...]) + fus_b[...]


def _enc_apply(fire, weath, topo, encw, fus_w, fus_b, s1, s2):
    args = (fire, weath, topo) + tuple(encw) + (fus_w, fus_b, s1, s2)
    return pl.pallas_call(
        _encapply_body,
        grid=(GRID,),
        in_specs=[pl.BlockSpec((BLK, T, 10), lambda i: (i, 0, 0)),
                  pl.BlockSpec((BLK, T, 8), lambda i: (i, 0, 0)),
                  pl.BlockSpec((BLK, T, 9), lambda i: (i, 0, 0))]
                 + [_full_spec(a) for a in args[3:]],
        out_specs=[pl.BlockSpec((BLK, HID), lambda i: (i, 0))] * T,
        out_shape=[jax.ShapeDtypeStruct((N, HID), jnp.float32)] * T,
    )(*args)


def _pre0_body(x, dinv, w, out):
    out[...] = _mmt(x[...], w[...]) * dinv[...][:, 0][:, None]


def _pre_l0(x, dinv, w):
    return pl.pallas_call(
        _pre0_body,
        grid=(GRID,),
        in_specs=[pl.BlockSpec((BLK, HID), lambda i: (i, 0)),
                  pl.BlockSpec((BLK, 1), lambda i: (i, 0)),
                  _full_spec(w)],
        out_specs=pl.BlockSpec((BLK, HID), lambda i: (i, 0)),
        out_shape=jax.ShapeDtypeStruct((N, HID), jnp.float32),
    )(x, dinv, w)


def _pre_body(x, s1, s2, gamma, beta, dinv, w, out):
    mean = s1[...][0] / N
    var = s2[...][0] / N - mean * mean
    scale = gamma[...] * lax.rsqrt(var + EPS)
    shift = beta[...] - mean * scale
    xt = _relu(x[...] * scale + shift)
    out[...] = _mmt(xt, w[...]) * dinv[...][:, 0][:, None]


def _pre_l(x, s1, s2, gamma, beta, dinv, w):
    args = (x, s1, s2, gamma, beta, dinv, w)
    return pl.pallas_call(
        _pre_body,
        grid=(GRID,),
        in_specs=[pl.BlockSpec((BLK, HID), lambda i: (i, 0)),
                  _full_spec(s1), _full_spec(s2), _full_spec(gamma),
                  _full_spec(beta),
                  pl.BlockSpec((BLK, 1), lambda i: (i, 0)),
                  _full_spec(w)],
        out_specs=pl.BlockSpec((BLK, HID), lambda i: (i, 0)),
        out_shape=jax.ShapeDtypeStruct((N, HID), jnp.float32),
    )(*args)


def _post_body(acc, gpre, dinv, b, out, s1, s2):
    i = pl.program_id(0)

    @pl.when(i == 0)
    def _init():
        s1[...] = jnp.zeros_like(s1)
        s2[...] = jnp.zeros_like(s2)

    dv = dinv[...][:, 0]
    a = jnp.concatenate([acc[0, :, :], acc[1, :, :]], axis=1)
    r = dv[:, None] * (a + gpre[...]) + b[...]
    out[...] = r
    s1[...] += jnp.sum(r, axis=0)[None]
    s2[...] += jnp.sum(r * r, axis=0)[None]


def _post_l(acc, gpre, dinv, b):
    return pl.pallas_call(
        _post_body,
        grid=(GRID,),
        in_specs=[pl.BlockSpec((2, BLK, 32), lambda i: (0, i, 0)),
                  pl.BlockSpec((BLK, HID), lambda i: (i, 0)),
                  pl.BlockSpec((BLK, 1), lambda i: (i, 0)),
                  _full_spec(b)],
        out_specs=[pl.BlockSpec((BLK, HID), lambda i: (i, 0)),
                   pl.BlockSpec((1, HID), lambda i: (0, 0)),
                   pl.BlockSpec((1, HID), lambda i: (0, 0))],
        out_shape=[jax.ShapeDtypeStruct((N, HID), jnp.float32),
                   jax.ShapeDtypeStruct((1, HID), jnp.float32),
                   jax.ShapeDtypeStruct((1, HID), jnp.float32)],
    )(acc, gpre, dinv, b)


def _lstm_body(x0, x1, x2, x3, s1_0, s2_0, s1_1, s2_1, s1_2, s2_2, s1_3, s2_3,
               gamma, beta,
               wih0, bih0, whh0, bhh0, wih1, bih1, whh1, bhh1,
               ow1, ob1, ow2, ob2, iw1, ib1, iw2, ib2,
               occ_out, int_out):
    gv = gamma[...]
    bv = beta[...]
    xs = []
    for xt, s1, s2 in ((x0, s1_0, s2_0), (x1, s1_1, s2_1),
                       (x2, s1_2, s2_2), (x3, s1_3, s2_3)):
        mean = s1[...][0] / N
        var = s2[...][0] / N - mean * mean
        scale = gv * lax.rsqrt(var + EPS)
        shift = bv - mean * scale
        xs.append(_relu(xt[...] * scale + shift))

    def lstm(seq, wih, bih, whh, bhh):
        h = jnp.zeros((BLK, HID), jnp.float32)
        cc = jnp.zeros((BLK, HID), jnp.float32)
        hs = []
        for t in range(T):
            gates = (_mmt(seq[t], wih[...]) + bih[...]
                     + _mmt(h, whh[...]) + bhh[...])
            ig = _sigmoid(gates[:, 0:HID])
            fg = _sigmoid(gates[:, HID:2 * HID])
            gg = jnp.tanh(gates[:, 2 * HID:3 * HID])
            og = _sigmoid(gates[:, 3 * HID:4 * HID])
            cc = fg * cc + ig * gg
            h = og * jnp.tanh(cc)
            hs.append(h)
        return hs

    h1 = lstm(xs, wih0, bih0, whh0, bhh0)
    h2 = lstm(h1, wih1, bih1, whh1, bhh1)
    final = h2[-1]
    ho = _relu(_mmt(final, ow1[...]) + ob1[...])
    occ = _sigmoid(jnp.sum(ho * ow2[...][0][None, :], axis=1, keepdims=True)
                   + ob2[...][0])
    hi = _relu(_mmt(final, iw1[...]) + ib1[...])
    inten = (jnp.sum(hi * iw2[...][0][None, :], axis=1, keepdims=True)
             + ib2[...][0])
    occ_out[...] = occ
    int_out[...] = inten


def _lstm_heads(xs, stats, gamma, beta, lstm_w, head_w):
    args = tuple(xs) + tuple(stats) + (gamma, beta) + tuple(lstm_w) + tuple(head_w)
    return pl.pallas_call(
        _lstm_body,
        grid=(GRID,),
        in_specs=[pl.BlockSpec((BLK, HID), lambda i: (i, 0))] * T
                 + [_full_spec(a) for a in args[T:]],
        out_specs=[pl.BlockSpec((BLK, 1), lambda i: (i, 0))] * 2,
        out_shape=[jax.ShapeDtypeStruct((N, 1), jnp.float32)] * 2,
    )(*args)


def _dinv_body(degs, out):
    d = degs[0, :, 0] + degs[1, :, 0] + 1.0
    out[...] = lax.rsqrt(d)[:, None]


def _dinv_tc(degs):
    return pl.pallas_call(
        _dinv_body,
        grid=(GRID,),
        in_specs=[pl.BlockSpec((2, BLK, 32), lambda i: (0, i, 0))],
        out_specs=pl.BlockSpec((BLK, 1), lambda i: (i, 0)),
        out_shape=jax.ShapeDtypeStruct((N, 1), jnp.float32),
    )(degs)


def kernel(fire_features, weather_features, topo_features, edge_index,
           fw1, fb1, fg1, fbe1, fw2, fb2,
           ww1, wb1, wg1, wbe1, ww2, wb2,
           tw1, tb1, tg1, tbe1, tw2, tb2,
           fus_w, fus_b,
           g0_w, g0_b, g0_gamma, g0_beta,
           g1_w, g1_b, g1_gamma, g1_beta,
           g2_w, g2_b, g2_gamma, g2_beta,
           l0_wih, l0_bih, l0_whh, l0_bhh,
           l1_wih, l1_bih, l1_whh, l1_bhh,
           occ_w1, occ_b1, occ_w2, occ_b2,
           int_w1, int_b1, int_w2, int_b2):
    src = edge_index[0]
    dst = edge_index[1]
    pad = EPAD - E
    src_p = jnp.concatenate([src, jnp.zeros((pad,), jnp.int32)])
    dst_p = jnp.concatenate([dst, jnp.full((pad,), DUMMY, jnp.int32)])
    dst32 = dst_p.reshape(32, 28, 7, 128)
    src2 = (src_p * 2)[None, :] + jnp.arange(2, dtype=jnp.int32)[:, None]
    src2 = src2.reshape(2, 16, 98, 4, 128)
    dstb = jnp.broadcast_to(dst_p.reshape(16, 98, 4, 128)[None],
                            (2, 16, 98, 4, 128))
    cidx = jnp.stack([src2, dstb], axis=3)
    zeros_acc = jnp.zeros((ACC_R, 32), jnp.float32)
    ones_rows = jnp.ones((128, 32), jnp.float32)

    degs = _sc_degree(dst32, ones_rows, zeros_acc)
    dinv = _dinv_tc(degs)

    s1e, s2e = _enc_stats(fire_features, weather_features, topo_features,
                          fw1, fb1, ww1, wb1, tw1, tb1)
    encw = (fw1, fb1, fg1, fbe1, fw2, fb2,
            ww1, wb1, wg1, wbe1, ww2, wb2,
            tw1, tb1, tg1, tbe1, tw2, tb2)
    h0 = _enc_apply(fire_features, weather_features, topo_features,
                    encw, fus_w, fus_b, s1e, s2e)

    gws = (g0_w, g1_w, g2_w)
    gbs = (g0_b, g1_b, g2_b)
    ggammas = (g0_gamma, g1_gamma, g2_gamma)
    gbetas = (g0_beta, g1_beta, g2_beta)

    raw = list(h0)
    stats = [None] * T
    for l in range(3):
        gpres = []
        for t in range(T):
            if l == 0:
                gpres.append(_pre_l0(raw[t], dinv, gws[l]))
            else:
                s1t, s2t = stats[t]
                gpres.append(_pre_l(raw[t], s1t, s2t, ggammas[l - 1],
                                    gbetas[l - 1], dinv, gws[l]))
        accs = [_sc_spmm(gpres[t].reshape(N * 2, 32), cidx, zeros_acc)
                for t in range(T)]
        for t in range(T):
            raw[t], s1t, s2t = _post_l(accs[t], gpres[t], dinv, gbs[l])
            stats[t] = (s1t, s2t)

    lstm_w = (l0_wih, l0_bih, l0_whh, l0_bhh, l1_wih, l1_bih, l1_whh, l1_bhh)
    head_w = (occ_w1, occ_b1, occ_w2, occ_b2, int_w1, int_b1, int_w2, int_b2)
    flat_stats = [s for pair in stats for s in pair]
    occ, inten = _lstm_heads(raw, flat_stats, g2_gamma, g2_beta,
                             lstm_w, head_w)
    return occ, inten


# speedup vs baseline: 14.6313x; 1.1025x over previous
import functools

import jax
import jax.numpy as jnp
from jax import lax
from jax.experimental import pallas as pl
from jax.experimental.pallas import tpu as pltpu
from jax.experimental.pallas import tpu_sc as plsc

N = 50000
T = 4
E = 800000
HID = 64
EPAD = 802816
ACC_R = 50048
RPT = ACC_R // 16
DUMMY = N
BLK = 2000
GRID = N // BLK
EPS = 1e-5

_SC_CACHE = {}


def _sc_mesh():
    return plsc.VectorSubcoreMesh(core_axis_name="c", subcore_axis_name="s")


def _mmt(x, w):
    return lax.dot_general(x, w, (((1,), (1,)), ((), ())),
                           preferred_element_type=jnp.float32)


def _relu(x):
    return jnp.maximum(x, 0.0)


def _sigmoid(x):
    return 1.0 / (1.0 + jnp.exp(-x))


def _deg_kernel_body(dst_hbm, ones_hbm, zeros_hbm, out_hbm, idx_d, ones_v, acc):
    c = lax.axis_index("c")
    s = lax.axis_index("s")
    wid = s * 2 + c
    pltpu.sync_copy(zeros_hbm.at[pl.ds(s * RPT, RPT)], acc.at[pl.ds(s * RPT, RPT)])
    pltpu.sync_copy(ones_hbm, ones_v)
    plsc.subcore_barrier()

    @pl.loop(0, 28)
    def _blk(k):
        pltpu.sync_copy(dst_hbm.at[wid, k], idx_d)
        for j in range(7):
            pltpu.sync_copy(ones_v, acc.at[idx_d.at[j]], add=True)

    plsc.subcore_barrier()
    pltpu.sync_copy(acc.at[pl.ds(s * RPT, RPT)], out_hbm.at[c, pl.ds(s * RPT, RPT)])


def _spmm_kernel_body(table_hbm, cidx_hbm, zeros_hbm, out_hbm,
                      cidx_v, rows, acc, gsem, ssem):
    c = lax.axis_index("c")
    s = lax.axis_index("s")
    pltpu.sync_copy(zeros_hbm.at[pl.ds(s * RPT, RPT)],
                    acc.at[pl.ds(s * RPT, RPT)])
    plsc.subcore_barrier()

    @pl.loop(0, 49)
    def _blk(k):
        pltpu.sync_copy(cidx_hbm.at[c, s, k], cidx_v)
        hs = [pltpu.async_copy(table_hbm.at[cidx_v.at[0, j]],
                               rows.at[j], gsem.at[j])
              for j in range(4)]
        sh = [None] * 4
        for j in range(8):
            b = j % 4
            hs[b].wait()
            sh[b] = pltpu.async_copy(rows.at[b], acc.at[cidx_v.at[1, j]],
                                     ssem.at[b], add=True)
            if j < 4:
                sh[b].wait()
                hs[b] = pltpu.async_copy(table_hbm.at[cidx_v.at[0, j + 4]],
                                         rows.at[b], gsem.at[b])
        for j in range(4):
            sh[j].wait()

    plsc.subcore_barrier()
    pltpu.sync_copy(acc.at[pl.ds(s * RPT, RPT)],
                    out_hbm.at[c, pl.ds(s * RPT, RPT)])


def _sc_degree(dst32, ones_rows, zeros_acc):
    if "deg" not in _SC_CACHE:
        _SC_CACHE["deg"] = pl.kernel(
            _deg_kernel_body,
            out_type=jax.ShapeDtypeStruct((2, ACC_R, 32), jnp.float32),
            mesh=_sc_mesh(),
            scratch_types=[
                pltpu.VMEM((7, 128), jnp.int32),
                pltpu.VMEM((128, 32), jnp.float32),
                pltpu.VMEM_SHARED((ACC_R, 32), jnp.float32),
            ],
            compiler_params=pltpu.CompilerParams(use_tc_tiling_on_sc=False),
        )
    return _SC_CACHE["deg"](dst32, ones_rows, zeros_acc)


def _sc_spmm(table, cidx, zeros_acc):
    if "spmm" not in _SC_CACHE:
        _SC_CACHE["spmm"] = pl.kernel(
            _spmm_kernel_body,
            out_type=jax.ShapeDtypeStruct((2, ACC_R, 32), jnp.float32),
            mesh=_sc_mesh(),
            scratch_types=[
                pltpu.VMEM((2, 8, 128), jnp.int32),
                pltpu.VMEM((4, 128, 32), jnp.float32),
                pltpu.VMEM_SHARED((ACC_R, 32), jnp.float32),
                pltpu.SemaphoreType.DMA((4,)),
                pltpu.SemaphoreType.DMA((4,)),
            ],
            compiler_params=pltpu.CompilerParams(use_tc_tiling_on_sc=False),
        )
    return _SC_CACHE["spmm"](table, cidx, zeros_acc)


def _full_spec(a):
    nd = a.ndim
    return pl.BlockSpec(a.shape, lambda i, _nd=nd: (0,) * _nd)


def _encstats_body(fire, weath, topo, fw1, fb1, ww1, wb1, tw1, tb1, s1, s2):
    i = pl.program_id(0)

    @pl.when(i == 0)
    def _init():
        s1[...] = jnp.zeros_like(s1)
        s2[...] = jnp.zeros_like(s2)

    sa, sb = [], []
    for t in range(T):
        hf = _relu(_mmt(fire[:, t, :], fw1[...]) + fb1[...])
        hw = _relu(_mmt(weath[:, t, :], ww1[...]) + wb1[...])
        ht = _relu(_mmt(topo[:, t, :], tw1[...]) + tb1[...])
        cat = jnp.concatenate([hf, hw, ht], axis=1)
        sa.append(jnp.sum(cat, axis=0))
        sb.append(jnp.sum(cat * cat, axis=0))
    s1[...] += jnp.stack(sa)
    s2[...] += jnp.stack(sb)


def _enc_stats(fire, weath, topo, fw1, fb1, ww1, wb1, tw1, tb1):
    args = (fire, weath, topo, fw1, fb1, ww1, wb1, tw1, tb1)
    return pl.pallas_call(
        _encstats_body,
        grid=(GRID,),
        in_specs=[pl.BlockSpec((BLK, T, 10), lambda i: (i, 0, 0)),
                  pl.BlockSpec((BLK, T, 8), lambda i: (i, 0, 0)),
                  pl.BlockSpec((BLK, T, 9), lambda i: (i, 0, 0))]
                 + [_full_spec(a) for a in args[3:]],
        out_specs=[pl.BlockSpec((T, 96), lambda i: (0, 0))] * 2,
        out_shape=[jax.ShapeDtypeStruct((T, 96), jnp.float32)] * 2,
    )(*args)


def _encapply_body(fire, weath, topo,
                   fw1, fb1, fg1, fbe1, fw2, fb2,
                   ww1, wb1, wg1, wbe1, ww2, wb2,
                   tw1, tb1, tg1, tbe1, tw2, tb2,
                   fus_w, fus_b, s1, s2, *outs):
    s1v = s1[...]
    s2v = s2[...]
    gcat = jnp.concatenate([fg1[...], wg1[...], tg1[...]])
    bcat = jnp.concatenate([fbe1[...], wbe1[...], tbe1[...]])
    for t in range(T):
        hf = _relu(_mmt(fire[:, t, :], fw1[...]) + fb1[...])
        hw = _relu(_mmt(weath[:, t, :], ww1[...]) + wb1[...])
        ht = _relu(_mmt(topo[:, t, :], tw1[...]) + tb1[...])
        cat = jnp.concatenate([hf, hw, ht], axis=1)
        mean = s1v[t] / N
        var = s2v[t] / N - mean * mean
        scale = gcat * lax.rsqrt(var + EPS)
        shift = bcat - mean * scale
        xn = cat * scale + shift
        ef = _mmt(xn[:, 0:32], fw2[...]) + fb2[...]
        ew = _mmt(xn[:, 32:64], ww2[...]) + wb2[...]
        et = _mmt(xn[:, 64:96], tw2[...]) + tb2[...]
        cat2 = jnp.concatenate([ef, ew, et], axis=1)
        outs[t][...] = _mmt(cat2, fus_w[...]) + fus_b[...]


def _enc_apply(fire, weath, topo, encw, fus_w, fus_b, s1, s2):
    args = (fire, weath, topo) + tuple(encw) + (fus_w, fus_b, s1, s2)
    return pl.pallas_call(
        _encapply_body,
        grid=(GRID,),
        in_specs=[pl.BlockSpec((BLK, T, 10), lambda i: (i, 0, 0)),
                  pl.BlockSpec((BLK, T, 8), lambda i: (i, 0, 0)),
                  pl.BlockSpec((BLK, T, 9), lambda i: (i, 0, 0))]
                 + [_full_spec(a) for a in args[3:]],
        out_specs=[pl.BlockSpec((BLK, HID), lambda i: (i, 0))] * T,
        out_shape=[jax.ShapeDtypeStruct((N, HID), jnp.float32)] * T,
    )(*args)


def _pre0_body(x, dinv, w, out):
    out[...] = _mmt(x[...], w[...]) * dinv[...][:, 0][:, None]


def _pre_l0(x, dinv, w):
    return pl.pallas_call(
        _pre0_body,
        grid=(GRID,),
        in_specs=[pl.BlockSpec((BLK, HID), lambda i: (i, 0)),
                  pl.BlockSpec((BLK, 1), lambda i: (i, 0)),
                  _full_spec(w)],
        out_specs=pl.BlockSpec((BLK, HID), lambda i: (i, 0)),
        out_shape=jax.ShapeDtypeStruct((N, HID), jnp.float32),
    )(x, dinv, w)


def _pre_body(x, s1, s2, gamma, beta, dinv, w, out):
    mean = s1[...][0] / N
    var = s2[...][0] / N - mean * mean
    scale = gamma[...] * lax.rsqrt(var + EPS)
    shift = beta[...] - mean * scale
    xt = _relu(x[...] * scale + shift)
    out[...] = _mmt(xt, w[...]) * dinv[...][:, 0][:, None]


def _pre_l(x, s1, s2, gamma, beta, dinv, w):
    args = (x, s1, s2, gamma, beta, dinv, w)
    return pl.pallas_call(
        _pre_body,
        grid=(GRID,),
        in_specs=[pl.BlockSpec((BLK, HID), lambda i: (i, 0)),
                  _full_spec(s1), _full_spec(s2), _full_spec(gamma),
                  _full_spec(beta),
                  pl.BlockSpec((BLK, 1), lambda i: (i, 0)),
                  _full_spec(w)],
        out_specs=pl.BlockSpec((BLK, HID), lambda i: (i, 0)),
        out_shape=jax.ShapeDtypeStruct((N, HID), jnp.float32),
    )(*args)


def _post_body(acc, gpre, dinv, b, out, s1, s2):
    i = pl.program_id(0)

    @pl.when(i == 0)
    def _init():
        s1[...] = jnp.zeros_like(s1)
        s2[...] = jnp.zeros_like(s2)

    dv = dinv[...][:, 0]
    a = jnp.concatenate([acc[0, :, :], acc[1, :, :]], axis=1)
    r = dv[:, None] * (a + gpre[...]) + b[...]
    out[...] = r
    s1[...] += jnp.sum(r, axis=0)[None]
    s2[...] += jnp.sum(r * r, axis=0)[None]


def _post_l(acc, gpre, dinv, b):
    return pl.pallas_call(
        _post_body,
        grid=(GRID,),
        in_specs=[pl.BlockSpec((2, BLK, 32), lambda i: (0, i, 0)),
                  pl.BlockSpec((BLK, HID), lambda i: (i, 0)),
                  pl.BlockSpec((BLK, 1), lambda i: (i, 0)),
                  _full_spec(b)],
        out_specs=[pl.BlockSpec((BLK, HID), lambda i: (i, 0)),
                   pl.BlockSpec((1, HID), lambda i: (0, 0)),
                   pl.BlockSpec((1, HID), lambda i: (0, 0))],
        out_shape=[jax.ShapeDtypeStruct((N, HID), jnp.float32),
                   jax.ShapeDtypeStruct((1, HID), jnp.float32),
                   jax.ShapeDtypeStruct((1, HID), jnp.float32)],
    )(acc, gpre, dinv, b)


def _lstm_body(x0, x1, x2, x3, s1_0, s2_0, s1_1, s2_1, s1_2, s2_2, s1_3, s2_3,
               gamma, beta,
               wih0, bih0, whh0, bhh0, wih1, bih1, whh1, bhh1,
               ow1, ob1, ow2, ob2, iw1, ib1, iw2, ib2,
               occ_out, int_out):
    gv = gamma[...]
    bv = beta[...]
    xs = []
    for xt, s1, s2 in ((x0, s1_0, s2_0), (x1, s1_1, s2_1),
                       (x2, s1_2, s2_2), (x3, s1_3, s2_3)):
        mean = s1[...][0] / N
        var = s2[...][0] / N - mean * mean
        scale = gv * lax.rsqrt(var + EPS)
        shift = bv - mean * scale
        xs.append(_relu(xt[...] * scale + shift))

    def lstm(seq, wih, bih, whh, bhh):
        h = jnp.zeros((BLK, HID), jnp.float32)
        cc = jnp.zeros((BLK, HID), jnp.float32)
        hs = []
        for t in range(T):
            gates = (_mmt(seq[t], wih[...]) + bih[...]
                     + _mmt(h, whh[...]) + bhh[...])
            ig = _sigmoid(gates[:, 0:HID])
            fg = _sigmoid(gates[:, HID:2 * HID])
            gg = jnp.tanh(gates[:, 2 * HID:3 * HID])
            og = _sigmoid(gates[:, 3 * HID:4 * HID])
            cc = fg * cc + ig * gg
            h = og * jnp.tanh(cc)
            hs.append(h)
        return hs

    h1 = lstm(xs, wih0, bih0, whh0, bhh0)
    h2 = lstm(h1, wih1, bih1, whh1, bhh1)
    final = h2[-1]
    ho = _relu(_mmt(final, ow1[...]) + ob1[...])
    occ = _sigmoid(jnp.sum(ho * ow2[...][0][None, :], axis=1, keepdims=True)
                   + ob2[...][0])
    hi = _relu(_mmt(final, iw1[...]) + ib1[...])
    inten = (jnp.sum(hi * iw2[...][0][None, :], axis=1, keepdims=True)
             + ib2[...][0])
    occ_out[...] = occ
    int_out[...] = inten


def _lstm_heads(xs, stats, gamma, beta, lstm_w, head_w):
    args = tuple(xs) + tuple(stats) + (gamma, beta) + tuple(lstm_w) + tuple(head_w)
    return pl.pallas_call(
        _lstm_body,
        grid=(GRID,),
        in_specs=[pl.BlockSpec((BLK, HID), lambda i: (i, 0))] * T
                 + [_full_spec(a) for a in args[T:]],
        out_specs=[pl.BlockSpec((BLK, 1), lambda i: (i, 0))] * 2,
        out_shape=[jax.ShapeDtypeStruct((N, 1), jnp.float32)] * 2,
    )(*args)


def _dinv_body(degs, out):
    d = degs[0, :, 0] + degs[1, :, 0] + 1.0
    out[...] = lax.rsqrt(d)[:, None]


def _dinv_tc(degs):
    return pl.pallas_call(
        _dinv_body,
        grid=(GRID,),
        in_specs=[pl.BlockSpec((2, BLK, 32), lambda i: (0, i, 0))],
        out_specs=pl.BlockSpec((BLK, 1), lambda i: (i, 0)),
        out_shape=jax.ShapeDtypeStruct((N, 1), jnp.float32),
    )(degs)


def kernel(fire_features, weather_features, topo_features, edge_index,
           fw1, fb1, fg1, fbe1, fw2, fb2,
           ww1, wb1, wg1, wbe1, ww2, wb2,
           tw1, tb1, tg1, tbe1, tw2, tb2,
           fus_w, fus_b,
           g0_w, g0_b, g0_gamma, g0_beta,
           g1_w, g1_b, g1_gamma, g1_beta,
           g2_w, g2_b, g2_gamma, g2_beta,
           l0_wih, l0_bih, l0_whh, l0_bhh,
           l1_wih, l1_bih, l1_whh, l1_bhh,
           occ_w1, occ_b1, occ_w2, occ_b2,
           int_w1, int_b1, int_w2, int_b2):
    src = edge_index[0]
    dst = edge_index[1]
    pad = EPAD - E
    src_p = jnp.concatenate([src, jnp.zeros((pad,), jnp.int32)])
    dst_p = jnp.concatenate([dst, jnp.full((pad,), DUMMY, jnp.int32)])
    dst32 = dst_p.reshape(32, 28, 7, 128)
    src2 = (src_p * 2)[None, :] + jnp.arange(2, dtype=jnp.int32)[:, None]
    src2 = src2.reshape(2, 16, 49, 8, 128)
    dstb = jnp.broadcast_to(dst_p.reshape(16, 49, 8, 128)[None],
                            (2, 16, 49, 8, 128))
    cidx = jnp.stack([src2, dstb], axis=3)
    zeros_acc = jnp.zeros((ACC_R, 32), jnp.float32)
    ones_rows = jnp.ones((128, 32), jnp.float32)

    degs = _sc_degree(dst32, ones_rows, zeros_acc)
    dinv = _dinv_tc(degs)

    s1e, s2e = _enc_stats(fire_features, weather_features, topo_features,
                          fw1, fb1, ww1, wb1, tw1, tb1)
    encw = (fw1, fb1, fg1, fbe1, fw2, fb2,
            ww1, wb1, wg1, wbe1, ww2, wb2,
            tw1, tb1, tg1, tbe1, tw2, tb2)
    h0 = _enc_apply(fire_features, weather_features, topo_features,
                    encw, fus_w, fus_b, s1e, s2e)

    gws = (g0_w, g1_w, g2_w)
    gbs = (g0_b, g1_b, g2_b)
    ggammas = (g0_gamma, g1_gamma, g2_gamma)
    gbetas = (g0_beta, g1_beta, g2_beta)

    raw = list(h0)
    stats = [None] * T
    for l in range(3):
        gpres = []
        for t in range(T):
            if l == 0:
                gpres.append(_pre_l0(raw[t], dinv, gws[l]))
            else:
                s1t, s2t = stats[t]
                gpres.append(_pre_l(raw[t], s1t, s2t, ggammas[l - 1],
                                    gbetas[l - 1], dinv, gws[l]))
        accs = [_sc_spmm(gpres[t].reshape(N * 2, 32), cidx, zeros_acc)
                for t in range(T)]
        for t in range(T):
            raw[t], s1t, s2t = _post_l(accs[t], gpres[t], dinv, gbs[l])
            stats[t] = (s1t, s2t)

    lstm_w = (l0_wih, l0_bih, l0_whh, l0_bhh, l1_wih, l1_bih, l1_whh, l1_bhh)
    head_w = (occ_w1, occ_b1, occ_w2, occ_b2, int_w1, int_b1, int_w2, int_b2)
    flat_stats = [s for pair in stats for s in pair]
    occ, inten = _lstm_heads(raw, flat_stats, g2_gamma, g2_beta,
                             lstm_w, head_w)
    return occ, inten


# speedup vs baseline: 14.6751x; 1.0030x over previous
import functools

import jax
import jax.numpy as jnp
from jax import lax
from jax.experimental import pallas as pl
from jax.experimental.pallas import tpu as pltpu
from jax.experimental.pallas import tpu_sc as plsc

N = 50000
T = 4
E = 800000
HID = 64
EPAD = 802816
ACC_R = 50048
RPT = ACC_R // 16
DUMMY = N
BLK = 2000
GRID = N // BLK
EPS = 1e-5

_SC_CACHE = {}


def _sc_mesh():
    return plsc.VectorSubcoreMesh(core_axis_name="c", subcore_axis_name="s")


def _mmt(x, w):
    return lax.dot_general(x, w, (((1,), (1,)), ((), ())),
                           preferred_element_type=jnp.float32)


def _relu(x):
    return jnp.maximum(x, 0.0)


def _sigmoid(x):
    return 1.0 / (1.0 + jnp.exp(-x))


def _deg_kernel_body(dst_hbm, ones_hbm, zeros_hbm, out_hbm, idx_d, ones_v, acc):
    c = lax.axis_index("c")
    s = lax.axis_index("s")
    wid = s * 2 + c
    pltpu.sync_copy(zeros_hbm.at[pl.ds(s * RPT, RPT)], acc.at[pl.ds(s * RPT, RPT)])
    pltpu.sync_copy(ones_hbm, ones_v)
    plsc.subcore_barrier()

    @pl.loop(0, 28)
    def _blk(k):
        pltpu.sync_copy(dst_hbm.at[wid, k], idx_d)
        for j in range(7):
            pltpu.sync_copy(ones_v, acc.at[idx_d.at[j]], add=True)

    plsc.subcore_barrier()
    pltpu.sync_copy(acc.at[pl.ds(s * RPT, RPT)], out_hbm.at[c, pl.ds(s * RPT, RPT)])


def _spmm_kernel_body(table_hbm, cidx_hbm, zeros_hbm, out_hbm,
                      cidx_v, rows, acc, gsem, ssem, zsem):
    c = lax.axis_index("c")
    s = lax.axis_index("s")
    zh = pltpu.async_copy(zeros_hbm.at[pl.ds(s * RPT, RPT)],
                          acc.at[pl.ds(s * RPT, RPT)], zsem)

    @pl.loop(0, 49)
    def _blk(k):
        pltpu.sync_copy(cidx_hbm.at[c, s, k], cidx_v)
        hs = [pltpu.async_copy(table_hbm.at[cidx_v.at[0, j]],
                               rows.at[j], gsem.at[j])
              for j in range(4)]

        @pl.when(k == 0)
        def _zero_sync():
            zh.wait()
            plsc.subcore_barrier()

        sh = [None] * 4
        for j in range(8):
            b = j % 4
            hs[b].wait()
            sh[b] = pltpu.async_copy(rows.at[b], acc.at[cidx_v.at[1, j]],
                                     ssem.at[b], add=True)
            if j < 4:
                sh[b].wait()
                hs[b] = pltpu.async_copy(table_hbm.at[cidx_v.at[0, j + 4]],
                                         rows.at[b], gsem.at[b])
        for j in range(4):
            sh[j].wait()

    plsc.subcore_barrier()
    pltpu.sync_copy(acc.at[pl.ds(s * RPT, RPT)],
                    out_hbm.at[c, pl.ds(s * RPT, RPT)])


def _sc_degree(dst32, ones_rows, zeros_acc):
    if "deg" not in _SC_CACHE:
        _SC_CACHE["deg"] = pl.kernel(
            _deg_kernel_body,
            out_type=jax.ShapeDtypeStruct((2, ACC_R, 32), jnp.float32),
            mesh=_sc_mesh(),
            scratch_types=[
                pltpu.VMEM((7, 128), jnp.int32),
                pltpu.VMEM((128, 32), jnp.float32),
                pltpu.VMEM_SHARED((ACC_R, 32), jnp.float32),
            ],
            compiler_params=pltpu.CompilerParams(use_tc_tiling_on_sc=False),
        )
    return _SC_CACHE["deg"](dst32, ones_rows, zeros_acc)


def _sc_spmm(table, cidx, zeros_acc):
    if "spmm" not in _SC_CACHE:
        _SC_CACHE["spmm"] = pl.kernel(
            _spmm_kernel_body,
            out_type=jax.ShapeDtypeStruct((2, ACC_R, 32), jnp.float32),
            mesh=_sc_mesh(),
            scratch_types=[
                pltpu.VMEM((2, 8, 128), jnp.int32),
                pltpu.VMEM((4, 128, 32), jnp.float32),
                pltpu.VMEM_SHARED((ACC_R, 32), jnp.float32),
                pltpu.SemaphoreType.DMA((4,)),
                pltpu.SemaphoreType.DMA((4,)),
                pltpu.SemaphoreType.DMA,
            ],
            compiler_params=pltpu.CompilerParams(use_tc_tiling_on_sc=False),
        )
    return _SC_CACHE["spmm"](table, cidx, zeros_acc)


def _full_spec(a):
    nd = a.ndim
    return pl.BlockSpec(a.shape, lambda i, _nd=nd: (0,) * _nd)


def _encstats_body(fire, weath, topo, fw1, fb1, ww1, wb1, tw1, tb1, s1, s2):
    i = pl.program_id(0)

    @pl.when(i == 0)
    def _init():
        s1[...] = jnp.zeros_like(s1)
        s2[...] = jnp.zeros_like(s2)

    sa, sb = [], []
    for t in range(T):
        hf = _relu(_mmt(fire[:, t, :], fw1[...]) + fb1[...])
        hw = _relu(_mmt(weath[:, t, :], ww1[...]) + wb1[...])
        ht = _relu(_mmt(topo[:, t, :], tw1[...]) + tb1[...])
        cat = jnp.concatenate([hf, hw, ht], axis=1)
        sa.append(jnp.sum(cat, axis=0))
        sb.append(jnp.sum(cat * cat, axis=0))
    s1[...] += jnp.stack(sa)
    s2[...] += jnp.stack(sb)


def _enc_stats(fire, weath, topo, fw1, fb1, ww1, wb1, tw1, tb1):
    args = (fire, weath, topo, fw1, fb1, ww1, wb1, tw1, tb1)
    return pl.pallas_call(
        _encstats_body,
        grid=(GRID,),
        in_specs=[pl.BlockSpec((BLK, T, 10), lambda i: (i, 0, 0)),
                  pl.BlockSpec((BLK, T, 8), lambda i: (i, 0, 0)),
                  pl.BlockSpec((BLK, T, 9), lambda i: (i, 0, 0))]
                 + [_full_spec(a) for a in args[3:]],
        out_specs=[pl.BlockSpec((T, 96), lambda i: (0, 0))] * 2,
        out_shape=[jax.ShapeDtypeStruct((T, 96), jnp.float32)] * 2,
    )(*args)


def _encapply_body(fire, weath, topo,
                   fw1, fb1, fg1, fbe1, fw2, fb2,
                   ww1, wb1, wg1, wbe1, ww2, wb2,
                   tw1, tb1, tg1, tbe1, tw2, tb2,
                   fus_w, fus_b, s1, s2, *outs):
    s1v = s1[...]
    s2v = s2[...]
    gcat = jnp.concatenate([fg1[...], wg1[...], tg1[...]])
    bcat = jnp.concatenate([fbe1[...], wbe1[...], tbe1[...]])
    for t in range(T):
        hf = _relu(_mmt(fire[:, t, :], fw1[...]) + fb1[...])
        hw = _relu(_mmt(weath[:, t, :], ww1[...]) + wb1[...])
        ht = _relu(_mmt(topo[:, t, :], tw1[...]) + tb1[...])
        cat = jnp.concatenate([hf, hw, ht], axis=1)
        mean = s1v[t] / N
        var = s2v[t] / N - mean * mean
        scale = gcat * lax.rsqrt(var + EPS)
        shift = bcat - mean * scale
        xn = cat * scale + shift
        ef = _mmt(xn[:, 0:32], fw2[...]) + fb2[...]
        ew = _mmt(xn[:, 32:64], ww2[...]) + wb2[...]
        et = _mmt(xn[:, 64:96], tw2[...]) + tb2[...]
        cat2 = jnp.concatenate([ef, ew, et], axis=1)
        outs[t][...] = _mmt(cat2, fus_w[...]) + fus_b[...]


def _enc_apply(fire, weath, topo, encw, fus_w, fus_b, s1, s2):
    args = (fire, weath, topo) + tuple(encw) + (fus_w, fus_b, s1, s2)
    return pl.pallas_call(
        _encapply_body,
        grid=(GRID,),
        in_specs=[pl.BlockSpec((BLK, T, 10), lambda i: (i, 0, 0)),
                  pl.BlockSpec((BLK, T, 8), lambda i: (i, 0, 0)),
                  pl.BlockSpec((BLK, T, 9), lambda i: (i, 0, 0))]
                 + [_full_spec(a) for a in args[3:]],
        out_specs=[pl.BlockSpec((BLK, HID), lambda i: (i, 0))] * T,
        out_shape=[jax.ShapeDtypeStruct((N, HID), jnp.float32)] * T,
    )(*args)


def _pre0_body(x, dinv, w, out):
    out[...] = _mmt(x[...], w[...]) * dinv[...][:, 0][:, None]


def _pre_l0(x, dinv, w):
    return pl.pallas_call(
        _pre0_body,
        grid=(GRID,),
        in_specs=[pl.BlockSpec((BLK, HID), lambda i: (i, 0)),
                  pl.BlockSpec((BLK, 1), lambda i: (i, 0)),
                  _full_spec(w)],
        out_specs=pl.BlockSpec((BLK, HID), lambda i: (i, 0)),
        out_shape=jax.ShapeDtypeStruct((N, HID), jnp.float32),
    )(x, dinv, w)


def _pre_body(x, s1, s2, gamma, beta, dinv, w, out):
    mean = s1[...][0] / N
    var = s2[...][0] / N - mean * mean
    scale = gamma[...] * lax.rsqrt(var + EPS)
    shift = beta[...] - mean * scale
    xt = _relu(x[...] * scale + shift)
    out[...] = _mmt(xt, w[...]) * dinv[...][:, 0][:, None]


def _pre_l(x, s1, s2, gamma, beta, dinv, w):
    args = (x, s1, s2, gamma, beta, dinv, w)
    return pl.pallas_call(
        _pre_body,
        grid=(GRID,),
        in_specs=[pl.BlockSpec((BLK, HID), lambda i: (i, 0)),
                  _full_spec(s1), _full_spec(s2), _full_spec(gamma),
                  _full_spec(beta),
                  pl.BlockSpec((BLK, 1), lambda i: (i, 0)),
                  _full_spec(w)],
        out_specs=pl.BlockSpec((BLK, HID), lambda i: (i, 0)),
        out_shape=jax.ShapeDtypeStruct((N, HID), jnp.float32),
    )(*args)


def _post_body(acc, gpre, dinv, b, out, s1, s2):
    i = pl.program_id(0)

    @pl.when(i == 0)
    def _init():
        s1[...] = jnp.zeros_like(s1)
        s2[...] = jnp.zeros_like(s2)

    dv = dinv[...][:, 0]
    a = jnp.concatenate([acc[0, :, :], acc[1, :, :]], axis=1)
    r = dv[:, None] * (a + gpre[...]) + b[...]
    out[...] = r
    s1[...] += jnp.sum(r, axis=0)[None]
    s2[...] += jnp.sum(r * r, axis=0)[None]


def _post_l(acc, gpre, dinv, b):
    return pl.pallas_call(
        _post_body,
        grid=(GRID,),
        in_specs=[pl.BlockSpec((2, BLK, 32), lambda i: (0, i, 0)),
                  pl.BlockSpec((BLK, HID), lambda i: (i, 0)),
                  pl.BlockSpec((BLK, 1), lambda i: (i, 0)),
                  _full_spec(b)],
        out_specs=[pl.BlockSpec((BLK, HID), lambda i: (i, 0)),
                   pl.BlockSpec((1, HID), lambda i: (0, 0)),
                   pl.BlockSpec((1, HID), lambda i: (0, 0))],
        out_shape=[jax.ShapeDtypeStruct((N, HID), jnp.float32),
                   jax.ShapeDtypeStruct((1, HID), jnp.float32),
                   jax.ShapeDtypeStruct((1, HID), jnp.float32)],
    )(acc, gpre, dinv, b)


def _lstm_body(x0, x1, x2, x3, s1_0, s2_0, s1_1, s2_1, s1_2, s2_2, s1_3, s2_3,
               gamma, beta,
               wih0, bih0, whh0, bhh0, wih1, bih1, whh1, bhh1,
               ow1, ob1, ow2, ob2, iw1, ib1, iw2, ib2,
               occ_out, int_out):
    gv = gamma[...]
    bv = beta[...]
    xs = []
    for xt, s1, s2 in ((x0, s1_0, s2_0), (x1, s1_1, s2_1),
                       (x2, s1_2, s2_2), (x3, s1_3, s2_3)):
        mean = s1[...][0] / N
        var = s2[...][0] / N - mean * mean
        scale = gv * lax.rsqrt(var + EPS)
        shift = bv - mean * scale
        xs.append(_relu(xt[...] * scale + shift))

    def lstm(seq, wih, bih, whh, bhh):
        h = jnp.zeros((BLK, HID), jnp.float32)
        cc = jnp.zeros((BLK, HID), jnp.float32)
        hs = []
        for t in range(T):
            gates = (_mmt(seq[t], wih[...]) + bih[...]
                     + _mmt(h, whh[...]) + bhh[...])
            ig = _sigmoid(gates[:, 0:HID])
            fg = _sigmoid(gates[:, HID:2 * HID])
            gg = jnp.tanh(gates[:, 2 * HID:3 * HID])
            og = _sigmoid(gates[:, 3 * HID:4 * HID])
            cc = fg * cc + ig * gg
            h = og * jnp.tanh(cc)
            hs.append(h)
        return hs

    h1 = lstm(xs, wih0, bih0, whh0, bhh0)
    h2 = lstm(h1, wih1, bih1, whh1, bhh1)
    final = h2[-1]
    ho = _relu(_mmt(final, ow1[...]) + ob1[...])
    occ = _sigmoid(jnp.sum(ho * ow2[...][0][None, :], axis=1, keepdims=True)
                   + ob2[...][0])
    hi = _relu(_mmt(final, iw1[...]) + ib1[...])
    inten = (jnp.sum(hi * iw2[...][0][None, :], axis=1, keepdims=True)
             + ib2[...][0])
    occ_out[...] = occ
    int_out[...] = inten


def _lstm_heads(xs, stats, gamma, beta, lstm_w, head_w):
    args = tuple(xs) + tuple(stats) + (gamma, beta) + tuple(lstm_w) + tuple(head_w)
    return pl.pallas_call(
        _lstm_body,
        grid=(GRID,),
        in_specs=[pl.BlockSpec((BLK, HID), lambda i: (i, 0))] * T
                 + [_full_spec(a) for a in args[T:]],
        out_specs=[pl.BlockSpec((BLK, 1), lambda i: (i, 0))] * 2,
        out_shape=[jax.ShapeDtypeStruct((N, 1), jnp.float32)] * 2,
    )(*args)


def _dinv_body(degs, out):
    d = degs[0, :, 0] + degs[1, :, 0] + 1.0
    out[...] = lax.rsqrt(d)[:, None]


def _dinv_tc(degs):
    return pl.pallas_call(
        _dinv_body,
        grid=(GRID,),
        in_specs=[pl.BlockSpec((2, BLK, 32), lambda i: (0, i, 0))],
        out_specs=pl.BlockSpec((BLK, 1), lambda i: (i, 0)),
        out_shape=jax.ShapeDtypeStruct((N, 1), jnp.float32),
    )(degs)


def kernel(fire_features, weather_features, topo_features, edge_index,
           fw1, fb1, fg1, fbe1, fw2, fb2,
           ww1, wb1, wg1, wbe1, ww2, wb2,
           tw1, tb1, tg1, tbe1, tw2, tb2,
           fus_w, fus_b,
           g0_w, g0_b, g0_gamma, g0_beta,
           g1_w, g1_b, g1_gamma, g1_beta,
           g2_w, g2_b, g2_gamma, g2_beta,
           l0_wih, l0_bih, l0_whh, l0_bhh,
           l1_wih, l1_bih, l1_whh, l1_bhh,
           occ_w1, occ_b1, occ_w2, occ_b2,
           int_w1, int_b1, int_w2, int_b2):
    src = edge_index[0]
    dst = edge_index[1]
    pad = EPAD - E
    src_p = jnp.concatenate([src, jnp.zeros((pad,), jnp.int32)])
    dst_p = jnp.concatenate([dst, jnp.full((pad,), DUMMY, jnp.int32)])
    dst32 = dst_p.reshape(32, 28, 7, 128)
    src2 = (src_p * 2)[None, :] + jnp.arange(2, dtype=jnp.int32)[:, None]
    src2 = src2.reshape(2, 16, 49, 8, 128)
    dstb = jnp.broadcast_to(dst_p.reshape(16, 49, 8, 128)[None],
                            (2, 16, 49, 8, 128))
    cidx = jnp.stack([src2, dstb], axis=3)
    zeros_acc = jnp.zeros((ACC_R, 32), jnp.float32)
    ones_rows = jnp.ones((128, 32), jnp.float32)

    degs = _sc_degree(dst32, ones_rows, zeros_acc)
    dinv = _dinv_tc(degs)

    s1e, s2e = _enc_stats(fire_features, weather_features, topo_features,
                          fw1, fb1, ww1, wb1, tw1, tb1)
    encw = (fw1, fb1, fg1, fbe1, fw2, fb2,
            ww1, wb1, wg1, wbe1, ww2, wb2,
            tw1, tb1, tg1, tbe1, tw2, tb2)
    h0 = _enc_apply(fire_features, weather_features, topo_features,
                    encw, fus_w, fus_b, s1e, s2e)

    gws = (g0_w, g1_w, g2_w)
    gbs = (g0_b, g1_b, g2_b)
    ggammas = (g0_gamma, g1_gamma, g2_gamma)
    gbetas = (g0_beta, g1_beta, g2_beta)

    raw = list(h0)
    stats = [None] * T
    for l in range(3):
        gpres = []
        for t in range(T):
            if l == 0:
                gpres.append(_pre_l0(raw[t], dinv, gws[l]))
            else:
                s1t, s2t = stats[t]
                gpres.append(_pre_l(raw[t], s1t, s2t, ggammas[l - 1],
                                    gbetas[l - 1], dinv, gws[l]))
        accs = [_sc_spmm(gpres[t].reshape(N * 2, 32), cidx, zeros_acc)
                for t in range(T)]
        for t in range(T):
            raw[t], s1t, s2t = _post_l(accs[t], gpres[t], dinv, gbs[l])
            stats[t] = (s1t, s2t)

    lstm_w = (l0_wih, l0_bih, l0_whh, l0_bhh, l1_wih, l1_bih, l1_whh, l1_bhh)
    head_w = (occ_w1, occ_b1, occ_w2, occ_b2, int_w1, int_b1, int_w2, int_b2)
    flat_stats = [s for pair in stats for s in pair]
    occ, inten = _lstm_heads(raw, flat_stats, g2_gamma, g2_beta,
                             lstm_w, head_w)
    return occ, inten


# speedup vs baseline: 16.0113x; 1.0911x over previous
import functools

import jax
import jax.numpy as jnp
from jax import lax
from jax.experimental import pallas as pl
from jax.experimental.pallas import tpu as pltpu
from jax.experimental.pallas import tpu_sc as plsc

N = 50000
T = 4
E = 800000
HID = 64
EPAD = 802816
ACC_R = 50048
RPT = ACC_R // 16
DUMMY = N
BLK = 2000
GRID = N // BLK
EPS = 1e-5

_SC_CACHE = {}


def _sc_mesh():
    return plsc.VectorSubcoreMesh(core_axis_name="c", subcore_axis_name="s")


def _mmt(x, w):
    return lax.dot_general(x, w, (((1,), (1,)), ((), ())),
                           preferred_element_type=jnp.float32)


def _relu(x):
    return jnp.maximum(x, 0.0)


def _sigmoid(x):
    return 1.0 / (1.0 + jnp.exp(-x))


def _deg_kernel_body(dst_hbm, ones_hbm, zeros_hbm, out_hbm, idx_d, ones_v, acc):
    c = lax.axis_index("c")
    s = lax.axis_index("s")
    wid = s * 2 + c
    pltpu.sync_copy(zeros_hbm.at[pl.ds(s * RPT, RPT)], acc.at[pl.ds(s * RPT, RPT)])
    pltpu.sync_copy(ones_hbm, ones_v)
    plsc.subcore_barrier()

    @pl.loop(0, 28)
    def _blk(k):
        pltpu.sync_copy(dst_hbm.at[wid, k], idx_d)
        for j in range(7):
            pltpu.sync_copy(ones_v, acc.at[idx_d.at[j]], add=True)

    plsc.subcore_barrier()
    pltpu.sync_copy(acc.at[pl.ds(s * RPT, RPT)], out_hbm.at[c, pl.ds(s * RPT, RPT)])


def _spmm_kernel_body(table_hbm, cidx_hbm, zeros_hbm, out_hbm,
                      cidx_v, rows, acc, gsem, ssem, zsem):
    c = lax.axis_index("c")
    s = lax.axis_index("s")
    zh = pltpu.async_copy(zeros_hbm.at[pl.ds(s * RPT, RPT)],
                          acc.at[pl.ds(s * RPT, RPT)], zsem)

    @pl.loop(0, 49)
    def _blk(k):
        pltpu.sync_copy(cidx_hbm.at[c, s, k], cidx_v)
        hs = [pltpu.async_copy(table_hbm.at[cidx_v.at[0, j]],
                               rows.at[j], gsem.at[j])
              for j in range(4)]

        @pl.when(k == 0)
        def _zero_sync():
            zh.wait()
            plsc.subcore_barrier()

        sh = [None] * 4
        for j in range(8):
            b = j % 4
            hs[b].wait()
            sh[b] = pltpu.async_copy(rows.at[b], acc.at[cidx_v.at[1, j]],
                                     ssem.at[b], add=True)
            if j < 4:
                sh[b].wait()
                hs[b] = pltpu.async_copy(table_hbm.at[cidx_v.at[0, j + 4]],
                                         rows.at[b], gsem.at[b])
        for j in range(4):
            sh[j].wait()

    plsc.subcore_barrier()
    pltpu.sync_copy(acc.at[pl.ds(s * RPT, RPT)],
                    out_hbm.at[c, pl.ds(s * RPT, RPT)])


def _sc_degree(dst32, ones_rows, zeros_acc):
    if "deg" not in _SC_CACHE:
        _SC_CACHE["deg"] = pl.kernel(
            _deg_kernel_body,
            out_type=jax.ShapeDtypeStruct((2, ACC_R, 32), jnp.float32),
            mesh=_sc_mesh(),
            scratch_types=[
                pltpu.VMEM((7, 128), jnp.int32),
                pltpu.VMEM((128, 32), jnp.float32),
                pltpu.VMEM_SHARED((ACC_R, 32), jnp.float32),
            ],
            compiler_params=pltpu.CompilerParams(use_tc_tiling_on_sc=False),
        )
    return _SC_CACHE["deg"](dst32, ones_rows, zeros_acc)


def _sc_spmm(table, cidx, zeros_acc):
    if "spmm" not in _SC_CACHE:
        _SC_CACHE["spmm"] = pl.kernel(
            _spmm_kernel_body,
            out_type=jax.ShapeDtypeStruct((2, ACC_R, 32), jnp.float32),
            mesh=_sc_mesh(),
            scratch_types=[
                pltpu.VMEM((2, 8, 128), jnp.int32),
                pltpu.VMEM((4, 128, 32), jnp.float32),
                pltpu.VMEM_SHARED((ACC_R, 32), jnp.float32),
                pltpu.SemaphoreType.DMA((4,)),
                pltpu.SemaphoreType.DMA((4,)),
                pltpu.SemaphoreType.DMA,
            ],
            compiler_params=pltpu.CompilerParams(use_tc_tiling_on_sc=False),
        )
    return _SC_CACHE["spmm"](table, cidx, zeros_acc)


def _full_spec(a):
    nd = a.ndim
    return pl.BlockSpec(a.shape, lambda i, _nd=nd: (0,) * _nd)


def _encstats_body(fire, weath, topo, fw1, fb1, ww1, wb1, tw1, tb1, s1, s2):
    i = pl.program_id(0)

    @pl.when(i == 0)
    def _init():
        s1[...] = jnp.zeros_like(s1)
        s2[...] = jnp.zeros_like(s2)

    hf = _relu(_mmt(fire[...], fw1[...]) + fb1[...])
    hw = _relu(_mmt(weath[...], ww1[...]) + wb1[...])
    ht = _relu(_mmt(topo[...], tw1[...]) + tb1[...])
    cat = jnp.concatenate([hf, hw, ht], axis=1)
    s1[...] += jnp.sum(cat, axis=0)[None]
    s2[...] += jnp.sum(cat * cat, axis=0)[None]


def _enc_stats(fire, weath, topo, fw1, fb1, ww1, wb1, tw1, tb1):
    args = (fire, weath, topo, fw1, fb1, ww1, wb1, tw1, tb1)
    return pl.pallas_call(
        _encstats_body,
        grid=(GRID,),
        in_specs=[pl.BlockSpec((BLK, 10), lambda i: (i, 0)),
                  pl.BlockSpec((BLK, 8), lambda i: (i, 0)),
                  pl.BlockSpec((BLK, 9), lambda i: (i, 0))]
                 + [_full_spec(a) for a in args[3:]],
        out_specs=[pl.BlockSpec((1, 96), lambda i: (0, 0))] * 2,
        out_shape=[jax.ShapeDtypeStruct((1, 96), jnp.float32)] * 2,
    )(*args)


def _encapply_body(fire, weath, topo,
                   fw1, fb1, fg1, fbe1, fw2, fb2,
                   ww1, wb1, wg1, wbe1, ww2, wb2,
                   tw1, tb1, tg1, tbe1, tw2, tb2,
                   fus_w, fus_b, s1, s2, out):
    gcat = jnp.concatenate([fg1[...], wg1[...], tg1[...]])
    bcat = jnp.concatenate([fbe1[...], wbe1[...], tbe1[...]])
    hf = _relu(_mmt(fire[...], fw1[...]) + fb1[...])
    hw = _relu(_mmt(weath[...], ww1[...]) + wb1[...])
    ht = _relu(_mmt(topo[...], tw1[...]) + tb1[...])
    cat = jnp.concatenate([hf, hw, ht], axis=1)
    mean = s1[...][0] / N
    var = s2[...][0] / N - mean * mean
    scale = gcat * lax.rsqrt(var + EPS)
    shift = bcat - mean * scale
    xn = cat * scale + shift
    ef = _mmt(xn[:, 0:32], fw2[...]) + fb2[...]
    ew = _mmt(xn[:, 32:64], ww2[...]) + wb2[...]
    et = _mmt(xn[:, 64:96], tw2[...]) + tb2[...]
    cat2 = jnp.concatenate([ef, ew, et], axis=1)
    out[...] = _mmt(cat2, fus_w[...]) + fus_b[...]


def _enc_apply(fire, weath, topo, encw, fus_w, fus_b, s1, s2):
    args = (fire, weath, topo) + tuple(encw) + (fus_w, fus_b, s1, s2)
    return pl.pallas_call(
        _encapply_body,
        grid=(GRID,),
        in_specs=[pl.BlockSpec((BLK, 10), lambda i: (i, 0)),
                  pl.BlockSpec((BLK, 8), lambda i: (i, 0)),
                  pl.BlockSpec((BLK, 9), lambda i: (i, 0))]
                 + [_full_spec(a) for a in args[3:]],
        out_specs=pl.BlockSpec((BLK, HID), lambda i: (i, 0)),
        out_shape=jax.ShapeDtypeStruct((N, HID), jnp.float32),
    )(*args)


def _pre0_body(x, dinv, w, out):
    out[...] = _mmt(x[...], w[...]) * dinv[...][:, 0][:, None]


def _pre_l0(x, dinv, w):
    return pl.pallas_call(
        _pre0_body,
        grid=(GRID,),
        in_specs=[pl.BlockSpec((BLK, HID), lambda i: (i, 0)),
                  pl.BlockSpec((BLK, 1), lambda i: (i, 0)),
                  _full_spec(w)],
        out_specs=pl.BlockSpec((BLK, HID), lambda i: (i, 0)),
        out_shape=jax.ShapeDtypeStruct((N, HID), jnp.float32),
    )(x, dinv, w)


def _pre_body(x, s1, s2, gamma, beta, dinv, w, out):
    mean = s1[...][0] / N
    var = s2[...][0] / N - mean * mean
    scale = gamma[...] * lax.rsqrt(var + EPS)
    shift = beta[...] - mean * scale
    xt = _relu(x[...] * scale + shift)
    out[...] = _mmt(xt, w[...]) * dinv[...][:, 0][:, None]


def _pre_l(x, s1, s2, gamma, beta, dinv, w):
    args = (x, s1, s2, gamma, beta, dinv, w)
    return pl.pallas_call(
        _pre_body,
        grid=(GRID,),
        in_specs=[pl.BlockSpec((BLK, HID), lambda i: (i, 0)),
                  _full_spec(s1), _full_spec(s2), _full_spec(gamma),
                  _full_spec(beta),
                  pl.BlockSpec((BLK, 1), lambda i: (i, 0)),
                  _full_spec(w)],
        out_specs=pl.BlockSpec((BLK, HID), lambda i: (i, 0)),
        out_shape=jax.ShapeDtypeStruct((N, HID), jnp.float32),
    )(*args)


def _post_body(acc, gpre, dinv, b, out, s1, s2):
    i = pl.program_id(0)

    @pl.when(i == 0)
    def _init():
        s1[...] = jnp.zeros_like(s1)
        s2[...] = jnp.zeros_like(s2)

    dv = dinv[...][:, 0]
    a = jnp.concatenate([acc[0, :, :], acc[1, :, :]], axis=1)
    r = dv[:, None] * (a + gpre[...]) + b[...]
    out[...] = r
    s1[...] += jnp.sum(r, axis=0)[None]
    s2[...] += jnp.sum(r * r, axis=0)[None]


def _post_l(acc, gpre, dinv, b):
    return pl.pallas_call(
        _post_body,
        grid=(GRID,),
        in_specs=[pl.BlockSpec((2, BLK, 32), lambda i: (0, i, 0)),
                  pl.BlockSpec((BLK, HID), lambda i: (i, 0)),
                  pl.BlockSpec((BLK, 1), lambda i: (i, 0)),
                  _full_spec(b)],
        out_specs=[pl.BlockSpec((BLK, HID), lambda i: (i, 0)),
                   pl.BlockSpec((1, HID), lambda i: (0, 0)),
                   pl.BlockSpec((1, HID), lambda i: (0, 0))],
        out_shape=[jax.ShapeDtypeStruct((N, HID), jnp.float32),
                   jax.ShapeDtypeStruct((1, HID), jnp.float32),
                   jax.ShapeDtypeStruct((1, HID), jnp.float32)],
    )(acc, gpre, dinv, b)


def _lstm_body(x0, x1, x2, x3, s1_0, s2_0, s1_1, s2_1, s1_2, s2_2, s1_3, s2_3,
               gamma, beta,
               wih0, bih0, whh0, bhh0, wih1, bih1, whh1, bhh1,
               ow1, ob1, ow2, ob2, iw1, ib1, iw2, ib2,
               occ_out, int_out):
    gv = gamma[...]
    bv = beta[...]
    xs = []
    for xt, s1, s2 in ((x0, s1_0, s2_0), (x1, s1_1, s2_1),
                       (x2, s1_2, s2_2), (x3, s1_3, s2_3)):
        mean = s1[...][0] / N
        var = s2[...][0] / N - mean * mean
        scale = gv * lax.rsqrt(var + EPS)
        shift = bv - mean * scale
        xs.append(_relu(xt[...] * scale + shift))

    def lstm(seq, wih, bih, whh, bhh):
        h = jnp.zeros((BLK, HID), jnp.float32)
        cc = jnp.zeros((BLK, HID), jnp.float32)
        hs = []
        for t in range(T):
            gates = (_mmt(seq[t], wih[...]) + bih[...]
                     + _mmt(h, whh[...]) + bhh[...])
            ig = _sigmoid(gates[:, 0:HID])
            fg = _sigmoid(gates[:, HID:2 * HID])
            gg = jnp.tanh(gates[:, 2 * HID:3 * HID])
            og = _sigmoid(gates[:, 3 * HID:4 * HID])
            cc = fg * cc + ig * gg
            h = og * jnp.tanh(cc)
            hs.append(h)
        return hs

    h1 = lstm(xs, wih0, bih0, whh0, bhh0)
    h2 = lstm(h1, wih1, bih1, whh1, bhh1)
    final = h2[-1]
    ho = _relu(_mmt(final, ow1[...]) + ob1[...])
    occ = _sigmoid(jnp.sum(ho * ow2[...][0][None, :], axis=1, keepdims=True)
                   + ob2[...][0])
    hi = _relu(_mmt(final, iw1[...]) + ib1[...])
    inten = (jnp.sum(hi * iw2[...][0][None, :], axis=1, keepdims=True)
             + ib2[...][0])
    occ_out[...] = occ
    int_out[...] = inten


def _lstm_heads(xs, stats, gamma, beta, lstm_w, head_w):
    args = tuple(xs) + tuple(stats) + (gamma, beta) + tuple(lstm_w) + tuple(head_w)
    return pl.pallas_call(
        _lstm_body,
        grid=(GRID,),
        in_specs=[pl.BlockSpec((BLK, HID), lambda i: (i, 0))] * T
                 + [_full_spec(a) for a in args[T:]],
        out_specs=[pl.BlockSpec((BLK, 1), lambda i: (i, 0))] * 2,
        out_shape=[jax.ShapeDtypeStruct((N, 1), jnp.float32)] * 2,
    )(*args)


def _dinv_body(degs, out):
    d = degs[0, :, 0] + degs[1, :, 0] + 1.0
    out[...] = lax.rsqrt(d)[:, None]


def _dinv_tc(degs):
    return pl.pallas_call(
        _dinv_body,
        grid=(GRID,),
        in_specs=[pl.BlockSpec((2, BLK, 32), lambda i: (0, i, 0))],
        out_specs=pl.BlockSpec((BLK, 1), lambda i: (i, 0)),
        out_shape=jax.ShapeDtypeStruct((N, 1), jnp.float32),
    )(degs)


def kernel(fire_features, weather_features, topo_features, edge_index,
           fw1, fb1, fg1, fbe1, fw2, fb2,
           ww1, wb1, wg1, wbe1, ww2, wb2,
           tw1, tb1, tg1, tbe1, tw2, tb2,
           fus_w, fus_b,
           g0_w, g0_b, g0_gamma, g0_beta,
           g1_w, g1_b, g1_gamma, g1_beta,
           g2_w, g2_b, g2_gamma, g2_beta,
           l0_wih, l0_bih, l0_whh, l0_bhh,
           l1_wih, l1_bih, l1_whh, l1_bhh,
           occ_w1, occ_b1, occ_w2, occ_b2,
           int_w1, int_b1, int_w2, int_b2):
    src = edge_index[0]
    dst = edge_index[1]
    pad = EPAD - E
    src_p = jnp.concatenate([src, jnp.zeros((pad,), jnp.int32)])
    dst_p = jnp.concatenate([dst, jnp.full((pad,), DUMMY, jnp.int32)])
    dst32 = dst_p.reshape(32, 28, 7, 128)
    src2 = (src_p * 2)[None, :] + jnp.arange(2, dtype=jnp.int32)[:, None]
    src2 = src2.reshape(2, 16, 49, 8, 128)
    dstb = jnp.broadcast_to(dst_p.reshape(16, 49, 8, 128)[None],
                            (2, 16, 49, 8, 128))
    cidx = jnp.stack([src2, dstb], axis=3)
    zeros_acc = jnp.zeros((ACC_R, 32), jnp.float32)
    ones_rows = jnp.ones((128, 32), jnp.float32)

    degs = _sc_degree(dst32, ones_rows, zeros_acc)
    dinv = _dinv_tc(degs)

    encw = (fw1, fb1, fg1, fbe1, fw2, fb2,
            ww1, wb1, wg1, wbe1, ww2, wb2,
            tw1, tb1, tg1, tbe1, tw2, tb2)
    h0 = []
    for t in range(T):
        ft = fire_features[:, t, :]
        wt = weather_features[:, t, :]
        tt = topo_features[:, t, :]
        s1e, s2e = _enc_stats(ft, wt, tt, fw1, fb1, ww1, wb1, tw1, tb1)
        h0.append(_enc_apply(ft, wt, tt, encw, fus_w, fus_b, s1e, s2e))

    gws = (g0_w, g1_w, g2_w)
    gbs = (g0_b, g1_b, g2_b)
    ggammas = (g0_gamma, g1_gamma, g2_gamma)
    gbetas = (g0_beta, g1_beta, g2_beta)

    raw = list(h0)
    stats = [None] * T
    for l in range(3):
        gpres = []
        for t in range(T):
            if l == 0:
                gpres.append(_pre_l0(raw[t], dinv, gws[l]))
            else:
                s1t, s2t = stats[t]
                gpres.append(_pre_l(raw[t], s1t, s2t, ggammas[l - 1],
                                    gbetas[l - 1], dinv, gws[l]))
        accs = [_sc_spmm(gpres[t].reshape(N * 2, 32), cidx, zeros_acc)
                for t in range(T)]
        for t in range(T):
            raw[t], s1t, s2t = _post_l(accs[t], gpres[t], dinv, gbs[l])
            stats[t] = (s1t, s2t)

    lstm_w = (l0_wih, l0_bih, l0_whh, l0_bhh, l1_wih, l1_bih, l1_whh, l1_bhh)
    head_w = (occ_w1, occ_b1, occ_w2, occ_b2, int_w1, int_b1, int_w2, int_b2)
    flat_stats = [s for pair in stats for s in pair]
    occ, inten = _lstm_heads(raw, flat_stats, g2_gamma, g2_beta,
                             lstm_w, head_w)
    return occ, inten


# speedup vs baseline: 16.0570x; 1.0029x over previous
import functools

import jax
import jax.numpy as jnp
from jax import lax
from jax.experimental import pallas as pl
from jax.experimental.pallas import tpu as pltpu
from jax.experimental.pallas import tpu_sc as plsc

N = 50000
T = 4
E = 800000
HID = 64
EPAD = 802816
ACC_R = 50048
RPT = ACC_R // 16
DUMMY = N
BLK = 2000
GRID = N // BLK
EPS = 1e-5

_SC_CACHE = {}


def _sc_mesh():
    return plsc.VectorSubcoreMesh(core_axis_name="c", subcore_axis_name="s")


def _mmt(x, w):
    return lax.dot_general(x, w, (((1,), (1,)), ((), ())),
                           preferred_element_type=jnp.float32)


def _relu(x):
    return jnp.maximum(x, 0.0)


def _sigmoid(x):
    return 1.0 / (1.0 + jnp.exp(-x))


def _deg_kernel_body(dst_hbm, ones_hbm, zeros_hbm, out_hbm, idx_d, ones_v, acc):
    c = lax.axis_index("c")
    s = lax.axis_index("s")
    wid = s * 2 + c
    pltpu.sync_copy(zeros_hbm.at[pl.ds(s * RPT, RPT)], acc.at[pl.ds(s * RPT, RPT)])
    pltpu.sync_copy(ones_hbm, ones_v)
    plsc.subcore_barrier()

    @pl.loop(0, 28)
    def _blk(k):
        pltpu.sync_copy(dst_hbm.at[wid, k], idx_d)
        for j in range(7):
            pltpu.sync_copy(ones_v, acc.at[idx_d.at[j]], add=True)

    plsc.subcore_barrier()
    pltpu.sync_copy(acc.at[pl.ds(s * RPT, RPT)], out_hbm.at[c, pl.ds(s * RPT, RPT)])


def _spmm_kernel_body(table_hbm, cidx_hbm, zeros_hbm, out_hbm,
                      cidx_v, rows, acc, gsem, ssem, zsem):
    c = lax.axis_index("c")
    s = lax.axis_index("s")
    zh = pltpu.async_copy(zeros_hbm.at[pl.ds(s * RPT, RPT)],
                          acc.at[pl.ds(s * RPT, RPT)], zsem)

    @pl.loop(0, 49)
    def _blk(k):
        pltpu.sync_copy(cidx_hbm.at[c, s, k], cidx_v)
        hs = [pltpu.async_copy(table_hbm.at[cidx_v.at[0, j]],
                               rows.at[j], gsem.at[j])
              for j in range(4)]

        @pl.when(k == 0)
        def _zero_sync():
            zh.wait()
            plsc.subcore_barrier()

        sh = [None] * 4
        for j in range(8):
            b = j % 4
            hs[b].wait()
            sh[b] = pltpu.async_copy(rows.at[b], acc.at[cidx_v.at[1, j]],
                                     ssem.at[b], add=True)
            if j < 4:
                sh[b].wait()
                hs[b] = pltpu.async_copy(table_hbm.at[cidx_v.at[0, j + 4]],
                                         rows.at[b], gsem.at[b])
        for j in range(4):
            sh[j].wait()

    plsc.subcore_barrier()
    pltpu.sync_copy(acc.at[pl.ds(s * RPT, RPT)],
                    out_hbm.at[c, pl.ds(s * RPT, RPT)])


def _sc_degree(dst32, ones_rows, zeros_acc):
    if "deg" not in _SC_CACHE:
        _SC_CACHE["deg"] = pl.kernel(
            _deg_kernel_body,
            out_type=jax.ShapeDtypeStruct((2, ACC_R, 32), jnp.float32),
            mesh=_sc_mesh(),
            scratch_types=[
                pltpu.VMEM((7, 128), jnp.int32),
                pltpu.VMEM((128, 32), jnp.float32),
                pltpu.VMEM_SHARED((ACC_R, 32), jnp.float32),
            ],
            compiler_params=pltpu.CompilerParams(use_tc_tiling_on_sc=False),
        )
    return _SC_CACHE["deg"](dst32, ones_rows, zeros_acc)


def _sc_spmm(table, cidx, zeros_acc):
    if "spmm" not in _SC_CACHE:
        _SC_CACHE["spmm"] = pl.kernel(
            _spmm_kernel_body,
            out_type=jax.ShapeDtypeStruct((2, ACC_R, 32), jnp.float32),
            mesh=_sc_mesh(),
            scratch_types=[
                pltpu.VMEM((2, 8, 128), jnp.int32),
                pltpu.VMEM((4, 128, 32), jnp.float32),
                pltpu.VMEM_SHARED((ACC_R, 32), jnp.float32),
                pltpu.SemaphoreType.DMA((4,)),
                pltpu.SemaphoreType.DMA((4,)),
                pltpu.SemaphoreType.DMA,
            ],
            compiler_params=pltpu.CompilerParams(use_tc_tiling_on_sc=False),
        )
    return _SC_CACHE["spmm"](table, cidx, zeros_acc)


def _full_spec(a):
    nd = a.ndim
    return pl.BlockSpec(a.shape, lambda i, _nd=nd: (0,) * _nd)


def _encstats_body(fire, weath, topo, fw1, fb1, ww1, wb1, tw1, tb1, s1, s2):
    i = pl.program_id(0)

    @pl.when(i == 0)
    def _init():
        s1[...] = jnp.zeros_like(s1)
        s2[...] = jnp.zeros_like(s2)

    hf = _relu(_mmt(fire[...], fw1[...]) + fb1[...])
    hw = _relu(_mmt(weath[...], ww1[...]) + wb1[...])
    ht = _relu(_mmt(topo[...], tw1[...]) + tb1[...])
    cat = jnp.concatenate([hf, hw, ht], axis=1)
    s1[...] += jnp.sum(cat, axis=0)[None]
    s2[...] += jnp.sum(cat * cat, axis=0)[None]


def _enc_stats(fire, weath, topo, fw1, fb1, ww1, wb1, tw1, tb1):
    args = (fire, weath, topo, fw1, fb1, ww1, wb1, tw1, tb1)
    return pl.pallas_call(
        _encstats_body,
        grid=(GRID,),
        in_specs=[pl.BlockSpec((BLK, 10), lambda i: (i, 0)),
                  pl.BlockSpec((BLK, 8), lambda i: (i, 0)),
                  pl.BlockSpec((BLK, 9), lambda i: (i, 0))]
                 + [_full_spec(a) for a in args[3:]],
        out_specs=[pl.BlockSpec((1, 96), lambda i: (0, 0))] * 2,
        out_shape=[jax.ShapeDtypeStruct((1, 96), jnp.float32)] * 2,
    )(*args)


def _encapply_body(fire, weath, topo,
                   fw1, fb1, fg1, fbe1, fw2, fb2,
                   ww1, wb1, wg1, wbe1, ww2, wb2,
                   tw1, tb1, tg1, tbe1, tw2, tb2,
                   fus_w, fus_b, s1, s2, out):
    gcat = jnp.concatenate([fg1[...], wg1[...], tg1[...]])
    bcat = jnp.concatenate([fbe1[...], wbe1[...], tbe1[...]])
    hf = _relu(_mmt(fire[...], fw1[...]) + fb1[...])
    hw = _relu(_mmt(weath[...], ww1[...]) + wb1[...])
    ht = _relu(_mmt(topo[...], tw1[...]) + tb1[...])
    cat = jnp.concatenate([hf, hw, ht], axis=1)
    mean = s1[...][0] / N
    var = s2[...][0] / N - mean * mean
    scale = gcat * lax.rsqrt(var + EPS)
    shift = bcat - mean * scale
    xn = cat * scale + shift
    ef = _mmt(xn[:, 0:32], fw2[...]) + fb2[...]
    ew = _mmt(xn[:, 32:64], ww2[...]) + wb2[...]
    et = _mmt(xn[:, 64:96], tw2[...]) + tb2[...]
    cat2 = jnp.concatenate([ef, ew, et], axis=1)
    out[...] = _mmt(cat2, fus_w[...]) + fus_b[...]


def _enc_apply(fire, weath, topo, encw, fus_w, fus_b, s1, s2):
    args = (fire, weath, topo) + tuple(encw) + (fus_w, fus_b, s1, s2)
    return pl.pallas_call(
        _encapply_body,
        grid=(GRID,),
        in_specs=[pl.BlockSpec((BLK, 10), lambda i: (i, 0)),
                  pl.BlockSpec((BLK, 8), lambda i: (i, 0)),
                  pl.BlockSpec((BLK, 9), lambda i: (i, 0))]
                 + [_full_spec(a) for a in args[3:]],
        out_specs=pl.BlockSpec((BLK, HID), lambda i: (i, 0)),
        out_shape=jax.ShapeDtypeStruct((N, HID), jnp.float32),
    )(*args)


def _pre0_body(x, dinv, w, out):
    out[...] = _mmt(x[...], w[...]) * dinv[...][:, 0][:, None]


def _pre_l0(x, dinv, w):
    return pl.pallas_call(
        _pre0_body,
        grid=(GRID,),
        in_specs=[pl.BlockSpec((BLK, HID), lambda i: (i, 0)),
                  pl.BlockSpec((BLK, 1), lambda i: (i, 0)),
                  _full_spec(w)],
        out_specs=pl.BlockSpec((BLK, HID), lambda i: (i, 0)),
        out_shape=jax.ShapeDtypeStruct((N, HID), jnp.float32),
    )(x, dinv, w)


def _pre_body(x, s1, s2, gamma, beta, dinv, w, out):
    mean = s1[...][0] / N
    var = s2[...][0] / N - mean * mean
    scale = gamma[...] * lax.rsqrt(var + EPS)
    shift = beta[...] - mean * scale
    xt = _relu(x[...] * scale + shift)
    out[...] = _mmt(xt, w[...]) * dinv[...][:, 0][:, None]


def _pre_l(x, s1, s2, gamma, beta, dinv, w):
    args = (x, s1, s2, gamma, beta, dinv, w)
    return pl.pallas_call(
        _pre_body,
        grid=(GRID,),
        in_specs=[pl.BlockSpec((BLK, HID), lambda i: (i, 0)),
                  _full_spec(s1), _full_spec(s2), _full_spec(gamma),
                  _full_spec(beta),
                  pl.BlockSpec((BLK, 1), lambda i: (i, 0)),
                  _full_spec(w)],
        out_specs=pl.BlockSpec((BLK, HID), lambda i: (i, 0)),
        out_shape=jax.ShapeDtypeStruct((N, HID), jnp.float32),
    )(*args)


def _post_body(acc, gpre, dinv, b, out, s1, s2):
    i = pl.program_id(0)

    @pl.when(i == 0)
    def _init():
        s1[...] = jnp.zeros_like(s1)
        s2[...] = jnp.zeros_like(s2)

    dv = dinv[...][:, 0]
    a = jnp.concatenate([acc[0, :, :], acc[1, :, :]], axis=1)
    r = dv[:, None] * (a + gpre[...]) + b[...]
    out[...] = r
    s1[...] += jnp.sum(r, axis=0)[None]
    s2[...] += jnp.sum(r * r, axis=0)[None]


def _post_l(acc, gpre, dinv, b):
    return pl.pallas_call(
        _post_body,
        grid=(GRID,),
        in_specs=[pl.BlockSpec((2, BLK, 32), lambda i: (0, i, 0)),
                  pl.BlockSpec((BLK, HID), lambda i: (i, 0)),
                  pl.BlockSpec((BLK, 1), lambda i: (i, 0)),
                  _full_spec(b)],
        out_specs=[pl.BlockSpec((BLK, HID), lambda i: (i, 0)),
                   pl.BlockSpec((1, HID), lambda i: (0, 0)),
                   pl.BlockSpec((1, HID), lambda i: (0, 0))],
        out_shape=[jax.ShapeDtypeStruct((N, HID), jnp.float32),
                   jax.ShapeDtypeStruct((1, HID), jnp.float32),
                   jax.ShapeDtypeStruct((1, HID), jnp.float32)],
    )(acc, gpre, dinv, b)


def _lstm_body(x0, x1, x2, x3, s1_0, s2_0, s1_1, s2_1, s1_2, s2_2, s1_3, s2_3,
               gamma, beta,
               wih0, bih0, whh0, bhh0, wih1, bih1, whh1, bhh1,
               ow1, ob1, ow2, ob2, iw1, ib1, iw2, ib2,
               occ_out, int_out):
    gv = gamma[...]
    bv = beta[...]
    xs = []
    for xt, s1, s2 in ((x0, s1_0, s2_0), (x1, s1_1, s2_1),
                       (x2, s1_2, s2_2), (x3, s1_3, s2_3)):
        mean = s1[...][0] / N
        var = s2[...][0] / N - mean * mean
        scale = gv * lax.rsqrt(var + EPS)
        shift = bv - mean * scale
        xs.append(_relu(xt[...] * scale + shift))

    def lstm(seq, wih, bih, whh, bhh):
        h = jnp.zeros((LBLK, HID), jnp.float32)
        cc = jnp.zeros((LBLK, HID), jnp.float32)
        hs = []
        for t in range(T):
            gates = (_mmt(seq[t], wih[...]) + bih[...]
                     + _mmt(h, whh[...]) + bhh[...])
            ig = _sigmoid(gates[:, 0:HID])
            fg = _sigmoid(gates[:, HID:2 * HID])
            gg = jnp.tanh(gates[:, 2 * HID:3 * HID])
            og = _sigmoid(gates[:, 3 * HID:4 * HID])
            cc = fg * cc + ig * gg
            h = og * jnp.tanh(cc)
            hs.append(h)
        return hs

    h1 = lstm(xs, wih0, bih0, whh0, bhh0)
    h2 = lstm(h1, wih1, bih1, whh1, bhh1)
    final = h2[-1]
    ho = _relu(_mmt(final, ow1[...]) + ob1[...])
    occ = _sigmoid(jnp.sum(ho * ow2[...][0][None, :], axis=1, keepdims=True)
                   + ob2[...][0])
    hi = _relu(_mmt(final, iw1[...]) + ib1[...])
    inten = (jnp.sum(hi * iw2[...][0][None, :], axis=1, keepdims=True)
             + ib2[...][0])
    occ_out[...] = occ
    int_out[...] = inten


LBLK = 5000


def _lstm_heads(xs, stats, gamma, beta, lstm_w, head_w):
    args = tuple(xs) + tuple(stats) + (gamma, beta) + tuple(lstm_w) + tuple(head_w)
    return pl.pallas_call(
        _lstm_body,
        grid=(N // LBLK,),
        in_specs=[pl.BlockSpec((LBLK, HID), lambda i: (i, 0))] * T
                 + [_full_spec(a) for a in args[T:]],
        out_specs=[pl.BlockSpec((LBLK, 1), lambda i: (i, 0))] * 2,
        out_shape=[jax.ShapeDtypeStruct((N, 1), jnp.float32)] * 2,
    )(*args)


def _dinv_body(degs, out):
    d = degs[0, :, 0] + degs[1, :, 0] + 1.0
    out[...] = lax.rsqrt(d)[:, None]


def _dinv_tc(degs):
    return pl.pallas_call(
        _dinv_body,
        grid=(GRID,),
        in_specs=[pl.BlockSpec((2, BLK, 32), lambda i: (0, i, 0))],
        out_specs=pl.BlockSpec((BLK, 1), lambda i: (i, 0)),
        out_shape=jax.ShapeDtypeStruct((N, 1), jnp.float32),
    )(degs)


def kernel(fire_features, weather_features, topo_features, edge_index,
           fw1, fb1, fg1, fbe1, fw2, fb2,
           ww1, wb1, wg1, wbe1, ww2, wb2,
           tw1, tb1, tg1, tbe1, tw2, tb2,
           fus_w, fus_b,
           g0_w, g0_b, g0_gamma, g0_beta,
           g1_w, g1_b, g1_gamma, g1_beta,
           g2_w, g2_b, g2_gamma, g2_beta,
           l0_wih, l0_bih, l0_whh, l0_bhh,
           l1_wih, l1_bih, l1_whh, l1_bhh,
           occ_w1, occ_b1, occ_w2, occ_b2,
           int_w1, int_b1, int_w2, int_b2):
    src = edge_index[0]
    dst = edge_index[1]
    pad = EPAD - E
    src_p = jnp.concatenate([src, jnp.zeros((pad,), jnp.int32)])
    dst_p = jnp.concatenate([dst, jnp.full((pad,), DUMMY, jnp.int32)])
    dst32 = dst_p.reshape(32, 28, 7, 128)
    src2 = (src_p * 2)[None, :] + jnp.arange(2, dtype=jnp.int32)[:, None]
    src2 = src2.reshape(2, 16, 49, 8, 128)
    dstb = jnp.broadcast_to(dst_p.reshape(16, 49, 8, 128)[None],
                            (2, 16, 49, 8, 128))
    cidx = jnp.stack([src2, dstb], axis=3)
    zeros_acc = jnp.zeros((ACC_R, 32), jnp.float32)
    ones_rows = jnp.ones((128, 32), jnp.float32)

    degs = _sc_degree(dst32, ones_rows, zeros_acc)
    dinv = _dinv_tc(degs)

    encw = (fw1, fb1, fg1, fbe1, fw2, fb2,
            ww1, wb1, wg1, wbe1, ww2, wb2,
            tw1, tb1, tg1, tbe1, tw2, tb2)
    h0 = []
    for t in range(T):
        ft = fire_features[:, t, :]
        wt = weather_features[:, t, :]
        tt = topo_features[:, t, :]
        s1e, s2e = _enc_stats(ft, wt, tt, fw1, fb1, ww1, wb1, tw1, tb1)
        h0.append(_enc_apply(ft, wt, tt, encw, fus_w, fus_b, s1e, s2e))

    gws = (g0_w, g1_w, g2_w)
    gbs = (g0_b, g1_b, g2_b)
    ggammas = (g0_gamma, g1_gamma, g2_gamma)
    gbetas = (g0_beta, g1_beta, g2_beta)

    raw = list(h0)
    stats = [None] * T
    for l in range(3):
        gpres = []
        for t in range(T):
            if l == 0:
                gpres.append(_pre_l0(raw[t], dinv, gws[l]))
            else:
                s1t, s2t = stats[t]
                gpres.append(_pre_l(raw[t], s1t, s2t, ggammas[l - 1],
                                    gbetas[l - 1], dinv, gws[l]))
        accs = [_sc_spmm(gpres[t].reshape(N * 2, 32), cidx, zeros_acc)
                for t in range(T)]
        for t in range(T):
            raw[t], s1t, s2t = _post_l(accs[t], gpres[t], dinv, gbs[l])
            stats[t] = (s1t, s2t)

    lstm_w = (l0_wih, l0_bih, l0_whh, l0_bhh, l1_wih, l1_bih, l1_whh, l1_bhh)
    head_w = (occ_w1, occ_b1, occ_w2, occ_b2, int_w1, int_b1, int_w2, int_b2)
    flat_stats = [s for pair in stats for s in pair]
    occ, inten = _lstm_heads(raw, flat_stats, g2_gamma, g2_beta,
                             lstm_w, head_w)
    return occ, inten
